# Initial kernel scaffold; baseline (speedup 1.0000x reference)
import jax
import jax.numpy as jnp
from jax.experimental import pallas as pl


def kernel(points_xyz, features, params):
    raise NotImplementedError("write your pallas kernel here")


# speedup vs baseline: 1.8520x; 1.8520x over previous
import functools

import jax
import jax.numpy as jnp
from jax.experimental import pallas as pl
from jax.experimental.pallas import tpu as pltpu

_B, _N, _F = 8, 4096, 13
_S = 1024
_RADII = (0.1, 0.2, 0.4)
_KS = (16, 32, 64)
_CIN = 16
_EPS = 1e-5
_BM = 2048


def _layer1_body(x_ref, c_ref, w_ref, b_ref, y_ref, st_ref, acc_ref, *,
                 k, nblocks, total):
    i = pl.program_id(0)
    rb = _BM // k
    x = x_ref[...]
    y = jnp.dot(x, w_ref[...], preferred_element_type=jnp.float32) + b_ref[...]
    corr = jnp.dot(c_ref[...], w_ref[0:3, :], preferred_element_type=jnp.float32)
    cout = y.shape[-1]
    corr = jnp.broadcast_to(corr[:, None, :], (rb, k, cout)).reshape(_BM, cout)
    y = y - corr
    y_ref[...] = y

    @pl.when(i == 0)
    def _():
        acc_ref[...] = jnp.zeros_like(acc_ref)

    acc_ref[0:1, :] += jnp.sum(y, axis=0, keepdims=True)
    acc_ref[1:2, :] += jnp.sum(y * y, axis=0, keepdims=True)

    @pl.when(i == nblocks - 1)
    def _():
        m = acc_ref[0:1, :] / total
        st_ref[0:1, :] = m
        st_ref[1:2, :] = acc_ref[1:2, :] / total - m * m


def _layer_body(x_ref, st_in_ref, g_ref, bt_ref, w_ref, b_ref,
                y_ref, st_ref, acc_ref, *, nblocks, total):
    i = pl.program_id(0)
    mu = st_in_ref[0:1, :]
    var = st_in_ref[1:2, :]
    a = g_ref[...] * (x_ref[...] - mu) * jax.lax.rsqrt(var + _EPS) + bt_ref[...]
    a = jnp.maximum(a, 0.0)
    y = jnp.dot(a, w_ref[...], preferred_element_type=jnp.float32) + b_ref[...]
    y_ref[...] = y

    @pl.when(i == 0)
    def _():
        acc_ref[...] = jnp.zeros_like(acc_ref)

    acc_ref[0:1, :] += jnp.sum(y, axis=0, keepdims=True)
    acc_ref[1:2, :] += jnp.sum(y * y, axis=0, keepdims=True)

    @pl.when(i == nblocks - 1)
    def _():
        m = acc_ref[0:1, :] / total
        st_ref[0:1, :] = m
        st_ref[1:2, :] = acc_ref[1:2, :] / total - m * m


def _final_body(x_ref, st_in_ref, g_ref, bt_ref, o_ref, *, k):
    mu = st_in_ref[0:1, :]
    var = st_in_ref[1:2, :]
    a = g_ref[...] * (x_ref[...] - mu) * jax.lax.rsqrt(var + _EPS) + bt_ref[...]
    a = jnp.maximum(a, 0.0)
    rb = _BM // k
    c = a.shape[-1]
    o_ref[...] = jnp.max(a.reshape(rb, k, c), axis=1)


def _mlp_scale(x_raw, cents, layers, k):
    m = x_raw.shape[0]
    nblocks = m // _BM
    rb = _BM // k
    f32 = jnp.float32

    (w1, b1, g1, t1), (w2, b2, g2, t2), (w3, b3, g3, t3) = layers
    c1, c2, c3 = w1.shape[0], w2.shape[0], w3.shape[0]
    w1t, w2t, w3t = w1.T, w2.T, w3.T

    row_spec = lambda c: pl.BlockSpec((_BM, c), lambda i: (i, 0))
    full = lambda a: pl.BlockSpec(a.shape, lambda i: (0,) * a.ndim)

    y1, st1 = pl.pallas_call(
        functools.partial(_layer1_body, k=k, nblocks=nblocks, total=float(m)),
        grid=(nblocks,),
        in_specs=[row_spec(_CIN), pl.BlockSpec((rb, 3), lambda i: (i, 0)),
                  full(w1t), pl.BlockSpec((1, c1), lambda i: (0, 0))],
        out_specs=[row_spec(c1), pl.BlockSpec((2, c1), lambda i: (0, 0))],
        out_shape=[jax.ShapeDtypeStruct((m, c1), f32),
                   jax.ShapeDtypeStruct((2, c1), f32)],
        scratch_shapes=[pltpu.VMEM((2, c1), f32)],
    )(x_raw, cents, w1t, b1.reshape(1, c1))

    def mid(y, st, g, bt, wt, b, cin, cout):
        return pl.pallas_call(
            functools.partial(_layer_body, nblocks=nblocks, total=float(m)),
            grid=(nblocks,),
            in_specs=[row_spec(cin), pl.BlockSpec((2, cin), lambda i: (0, 0)),
                      pl.BlockSpec((1, cin), lambda i: (0, 0)),
                      pl.BlockSpec((1, cin), lambda i: (0, 0)),
                      full(wt), pl.BlockSpec((1, cout), lambda i: (0, 0))],
            out_specs=[row_spec(cout), pl.BlockSpec((2, cout), lambda i: (0, 0))],
            out_shape=[jax.ShapeDtypeStruct((m, cout), f32),
                       jax.ShapeDtypeStruct((2, cout), f32)],
            scratch_shapes=[pltpu.VMEM((2, cout), f32)],
        )(y, st, g.reshape(1, cin), bt.reshape(1, cin), wt, b.reshape(1, cout))

    y2, st2 = mid(y1, st1, g1, t1, w2t, b2, c1, c2)
    y3, st3 = mid(y2, st2, g2, t2, w3t, b3, c2, c3)

    out = pl.pallas_call(
        functools.partial(_final_body, k=k),
        grid=(nblocks,),
        in_specs=[row_spec(c3), pl.BlockSpec((2, c3), lambda i: (0, 0)),
                  pl.BlockSpec((1, c3), lambda i: (0, 0)),
                  pl.BlockSpec((1, c3), lambda i: (0, 0))],
        out_specs=pl.BlockSpec((rb, c3), lambda i: (i, 0)),
        out_shape=jax.ShapeDtypeStruct((m // k, c3), f32),
    )(y3, st3, g3.reshape(1, c3), t3.reshape(1, c3))
    return out


def _fps_jax(xyz):
    b, n, _ = xyz.shape

    def body(i, state):
        dist, far = state
        centroid = jnp.take_along_axis(xyz, far[:, None, None], axis=1)
        d = jnp.sum((xyz - centroid) ** 2, axis=-1)
        dist = jnp.minimum(dist, d)
        far2 = jnp.argmax(dist, axis=-1).astype(jnp.int32)
        return dist, far2

    def scan_body(state, _):
        dist, far = state
        centroid = jnp.take_along_axis(xyz, far[:, None, None], axis=1)
        d = jnp.sum((xyz - centroid) ** 2, axis=-1)
        dist2 = jnp.minimum(dist, d)
        far2 = jnp.argmax(dist2, axis=-1).astype(jnp.int32)
        return (dist2, far2), centroid[:, 0, :]

    dist0 = jnp.full((b, n), 1e10, dtype=xyz.dtype)
    far0 = jnp.zeros((b,), dtype=jnp.int32)
    (_, _), cents = jax.lax.scan(scan_body, (dist0, far0), None, length=_S)
    return jnp.transpose(cents, (1, 0, 2))


def _ball_jax(xyz, cents, radius, k):
    b, n, _ = xyz.shape
    sqr = jnp.sum((cents[:, :, None, :] - xyz[:, None, :, :]) ** 2, axis=-1)
    idx = jnp.broadcast_to(jnp.arange(n, dtype=jnp.int32), sqr.shape)
    idx = jnp.where(sqr > radius * radius, n, idx)
    idx = jnp.sort(idx, axis=-1)[:, :, :k]
    first = idx[:, :, :1]
    return jnp.where(idx == n, jnp.broadcast_to(first, idx.shape), idx)


def kernel(points_xyz, features, params):
    cents = _fps_jax(points_xyz)
    table = jnp.concatenate([points_xyz, features], axis=-1)
    table_flat = table.reshape(_B * _N, _CIN)
    cents_flat = cents.reshape(_B * _S, 3)

    outs = []
    for radius, k, layers in zip(_RADII, _KS, params):
        gidx = _ball_jax(points_xyz, cents, radius, k)
        flat = (gidx + jnp.arange(_B, dtype=jnp.int32)[:, None, None] * _N)
        x_raw = table_flat[flat.reshape(-1)]
        out = _mlp_scale(x_raw, cents_flat, layers, k)
        outs.append(out.reshape(_B, _S, -1))
    return cents, jnp.concatenate(outs, axis=-1)


# speedup vs baseline: 2.5877x; 1.3972x over previous
import functools

import jax
import jax.numpy as jnp
from jax.experimental import pallas as pl
from jax.experimental.pallas import tpu as pltpu

_B, _N, _F = 8, 4096, 13
_S = 1024
_RADII = (0.1, 0.2, 0.4)
_KS = (16, 32, 64)
_CIN = 16
_EPS = 1e-5
_BM = 2048


def _layer1_body(x_ref, c_ref, w_ref, b_ref, y_ref, st_ref, acc_ref, *,
                 k, nblocks, total):
    i = pl.program_id(0)
    rb = _BM // k
    x = x_ref[...]
    y = jnp.dot(x, w_ref[...], preferred_element_type=jnp.float32) + b_ref[...]
    corr = jnp.dot(c_ref[...], w_ref[0:3, :], preferred_element_type=jnp.float32)
    cout = y.shape[-1]
    corr = jnp.broadcast_to(corr[:, None, :], (rb, k, cout)).reshape(_BM, cout)
    y = y - corr
    y_ref[...] = y

    @pl.when(i == 0)
    def _():
        acc_ref[...] = jnp.zeros_like(acc_ref)

    acc_ref[0:1, :] += jnp.sum(y, axis=0, keepdims=True)
    acc_ref[1:2, :] += jnp.sum(y * y, axis=0, keepdims=True)

    @pl.when(i == nblocks - 1)
    def _():
        m = acc_ref[0:1, :] / total
        st_ref[0:1, :] = m
        st_ref[1:2, :] = acc_ref[1:2, :] / total - m * m


def _layer_body(x_ref, st_in_ref, g_ref, bt_ref, w_ref, b_ref,
                y_ref, st_ref, acc_ref, *, nblocks, total):
    i = pl.program_id(0)
    mu = st_in_ref[0:1, :]
    var = st_in_ref[1:2, :]
    a = g_ref[...] * (x_ref[...] - mu) * jax.lax.rsqrt(var + _EPS) + bt_ref[...]
    a = jnp.maximum(a, 0.0)
    y = jnp.dot(a, w_ref[...], preferred_element_type=jnp.float32) + b_ref[...]
    y_ref[...] = y

    @pl.when(i == 0)
    def _():
        acc_ref[...] = jnp.zeros_like(acc_ref)

    acc_ref[0:1, :] += jnp.sum(y, axis=0, keepdims=True)
    acc_ref[1:2, :] += jnp.sum(y * y, axis=0, keepdims=True)

    @pl.when(i == nblocks - 1)
    def _():
        m = acc_ref[0:1, :] / total
        st_ref[0:1, :] = m
        st_ref[1:2, :] = acc_ref[1:2, :] / total - m * m


def _final_body(x_ref, st_in_ref, g_ref, bt_ref, o_ref, *, k):
    mu = st_in_ref[0:1, :]
    var = st_in_ref[1:2, :]
    a = g_ref[...] * (x_ref[...] - mu) * jax.lax.rsqrt(var + _EPS) + bt_ref[...]
    a = jnp.maximum(a, 0.0)
    rb = _BM // k
    c = a.shape[-1]
    o_ref[...] = jnp.max(a.reshape(rb, k, c), axis=1)


def _mlp_scale(x_raw, cents, layers, k):
    m = x_raw.shape[0]
    nblocks = m // _BM
    rb = _BM // k
    f32 = jnp.float32

    (w1, b1, g1, t1), (w2, b2, g2, t2), (w3, b3, g3, t3) = layers
    c1, c2, c3 = w1.shape[0], w2.shape[0], w3.shape[0]
    w1t, w2t, w3t = w1.T, w2.T, w3.T

    row_spec = lambda c: pl.BlockSpec((_BM, c), lambda i: (i, 0))
    full = lambda a: pl.BlockSpec(a.shape, lambda i: (0,) * a.ndim)

    y1, st1 = pl.pallas_call(
        functools.partial(_layer1_body, k=k, nblocks=nblocks, total=float(m)),
        grid=(nblocks,),
        in_specs=[row_spec(_CIN), pl.BlockSpec((rb, 3), lambda i: (i, 0)),
                  full(w1t), pl.BlockSpec((1, c1), lambda i: (0, 0))],
        out_specs=[row_spec(c1), pl.BlockSpec((2, c1), lambda i: (0, 0))],
        out_shape=[jax.ShapeDtypeStruct((m, c1), f32),
                   jax.ShapeDtypeStruct((2, c1), f32)],
        scratch_shapes=[pltpu.VMEM((2, c1), f32)],
    )(x_raw, cents, w1t, b1.reshape(1, c1))

    def mid(y, st, g, bt, wt, b, cin, cout):
        return pl.pallas_call(
            functools.partial(_layer_body, nblocks=nblocks, total=float(m)),
            grid=(nblocks,),
            in_specs=[row_spec(cin), pl.BlockSpec((2, cin), lambda i: (0, 0)),
                      pl.BlockSpec((1, cin), lambda i: (0, 0)),
                      pl.BlockSpec((1, cin), lambda i: (0, 0)),
                      full(wt), pl.BlockSpec((1, cout), lambda i: (0, 0))],
            out_specs=[row_spec(cout), pl.BlockSpec((2, cout), lambda i: (0, 0))],
            out_shape=[jax.ShapeDtypeStruct((m, cout), f32),
                       jax.ShapeDtypeStruct((2, cout), f32)],
            scratch_shapes=[pltpu.VMEM((2, cout), f32)],
        )(y, st, g.reshape(1, cin), bt.reshape(1, cin), wt, b.reshape(1, cout))

    y2, st2 = mid(y1, st1, g1, t1, w2t, b2, c1, c2)
    y3, st3 = mid(y2, st2, g2, t2, w3t, b3, c2, c3)

    out = pl.pallas_call(
        functools.partial(_final_body, k=k),
        grid=(nblocks,),
        in_specs=[row_spec(c3), pl.BlockSpec((2, c3), lambda i: (0, 0)),
                  pl.BlockSpec((1, c3), lambda i: (0, 0)),
                  pl.BlockSpec((1, c3), lambda i: (0, 0))],
        out_specs=pl.BlockSpec((rb, c3), lambda i: (i, 0)),
        out_shape=jax.ShapeDtypeStruct((m // k, c3), f32),
    )(y3, st3, g3.reshape(1, c3), t3.reshape(1, c3))
    return out


def _fps_body(x_ref, y_ref, z_ref, cx_ref, cy_ref, cz_ref):
    x = x_ref[...]
    y = y_ref[...]
    z = z_ref[...]
    iota_n = jax.lax.broadcasted_iota(jnp.int32, (_B, _N), 1)
    iota_s = jax.lax.broadcasted_iota(jnp.int32, (_B, _S), 1)
    cx_ref[...] = jnp.zeros_like(cx_ref)
    cy_ref[...] = jnp.zeros_like(cy_ref)
    cz_ref[...] = jnp.zeros_like(cz_ref)

    def body(i, carry):
        dist, far = carry
        oh = (iota_n == far).astype(jnp.float32)
        cx = jnp.sum(x * oh, axis=1, keepdims=True)
        cy = jnp.sum(y * oh, axis=1, keepdims=True)
        cz = jnp.sum(z * oh, axis=1, keepdims=True)
        sel = (iota_s == i).astype(jnp.float32)
        cx_ref[...] += cx * sel
        cy_ref[...] += cy * sel
        cz_ref[...] += cz * sel
        dx = x - cx
        dy = y - cy
        dz = z - cz
        d = dx * dx + dy * dy + dz * dz
        dist = jnp.minimum(dist, d)
        m = jnp.max(dist, axis=1, keepdims=True)
        far2 = jnp.min(jnp.where(dist == m, iota_n, _N), axis=1, keepdims=True)
        return dist, far2.astype(jnp.int32)

    dist0 = jnp.full((_B, _N), 1e10, jnp.float32)
    far0 = jnp.zeros((_B, 1), jnp.int32)
    jax.lax.fori_loop(0, _S, body, (dist0, far0))


def _fps_pallas(points_xyz):
    f32 = jnp.float32
    x = points_xyz[:, :, 0]
    y = points_xyz[:, :, 1]
    z = points_xyz[:, :, 2]
    cx, cy, cz = pl.pallas_call(
        _fps_body,
        out_shape=[jax.ShapeDtypeStruct((_B, _S), f32)] * 3,
    )(x, y, z)
    return jnp.stack([cx, cy, cz], axis=-1)


def _fps_jax(xyz):
    b, n, _ = xyz.shape

    def body(i, state):
        dist, far = state
        centroid = jnp.take_along_axis(xyz, far[:, None, None], axis=1)
        d = jnp.sum((xyz - centroid) ** 2, axis=-1)
        dist = jnp.minimum(dist, d)
        far2 = jnp.argmax(dist, axis=-1).astype(jnp.int32)
        return dist, far2

    def scan_body(state, _):
        dist, far = state
        centroid = jnp.take_along_axis(xyz, far[:, None, None], axis=1)
        d = jnp.sum((xyz - centroid) ** 2, axis=-1)
        dist2 = jnp.minimum(dist, d)
        far2 = jnp.argmax(dist2, axis=-1).astype(jnp.int32)
        return (dist2, far2), centroid[:, 0, :]

    dist0 = jnp.full((b, n), 1e10, dtype=xyz.dtype)
    far0 = jnp.zeros((b,), dtype=jnp.int32)
    (_, _), cents = jax.lax.scan(scan_body, (dist0, far0), None, length=_S)
    return jnp.transpose(cents, (1, 0, 2))


def _ball_jax(xyz, cents, radius, k):
    b, n, _ = xyz.shape
    sqr = jnp.sum((cents[:, :, None, :] - xyz[:, None, :, :]) ** 2, axis=-1)
    idx = jnp.broadcast_to(jnp.arange(n, dtype=jnp.int32), sqr.shape)
    idx = jnp.where(sqr > radius * radius, n, idx)
    idx = jnp.sort(idx, axis=-1)[:, :, :k]
    first = idx[:, :, :1]
    return jnp.where(idx == n, jnp.broadcast_to(first, idx.shape), idx)


def kernel(points_xyz, features, params):
    cents = _fps_pallas(points_xyz)
    table = jnp.concatenate([points_xyz, features], axis=-1)
    table_flat = table.reshape(_B * _N, _CIN)
    cents_flat = cents.reshape(_B * _S, 3)

    outs = []
    for radius, k, layers in zip(_RADII, _KS, params):
        gidx = _ball_jax(points_xyz, cents, radius, k)
        flat = (gidx + jnp.arange(_B, dtype=jnp.int32)[:, None, None] * _N)
        x_raw = table_flat[flat.reshape(-1)]
        out = _mlp_scale(x_raw, cents_flat, layers, k)
        outs.append(out.reshape(_B, _S, -1))
    return cents, jnp.concatenate(outs, axis=-1)


# speedup vs baseline: 9.8982x; 3.8251x over previous
import functools

import jax
import jax.numpy as jnp
from jax import lax
from jax.experimental import pallas as pl
from jax.experimental.pallas import tpu as pltpu
from jax.experimental.pallas import tpu_sc as plsc

_B, _N, _F = 8, 4096, 13
_S = 1024
_RADII = (0.1, 0.2, 0.4)
_KS = (16, 32, 64)
_CIN = 16
_EPS = 1e-5
_BM = 2048


def _layer1_body(x_ref, c_ref, w_ref, b_ref, y_ref, st_ref, acc_ref, *,
                 k, nblocks, total):
    i = pl.program_id(0)
    rb = _BM // k
    x = x_ref[...]
    y = jnp.dot(x, w_ref[...], preferred_element_type=jnp.float32) + b_ref[...]
    corr = jnp.dot(c_ref[...], w_ref[0:3, :], preferred_element_type=jnp.float32)
    cout = y.shape[-1]
    corr = jnp.broadcast_to(corr[:, None, :], (rb, k, cout)).reshape(_BM, cout)
    y = y - corr
    y_ref[...] = y

    @pl.when(i == 0)
    def _():
        acc_ref[...] = jnp.zeros_like(acc_ref)

    acc_ref[0:1, :] += jnp.sum(y, axis=0, keepdims=True)
    acc_ref[1:2, :] += jnp.sum(y * y, axis=0, keepdims=True)

    @pl.when(i == nblocks - 1)
    def _():
        m = acc_ref[0:1, :] / total
        st_ref[0:1, :] = m
        st_ref[1:2, :] = acc_ref[1:2, :] / total - m * m


def _layer_body(x_ref, st_in_ref, g_ref, bt_ref, w_ref, b_ref,
                y_ref, st_ref, acc_ref, *, nblocks, total):
    i = pl.program_id(0)
    mu = st_in_ref[0:1, :]
    var = st_in_ref[1:2, :]
    a = g_ref[...] * (x_ref[...] - mu) * jax.lax.rsqrt(var + _EPS) + bt_ref[...]
    a = jnp.maximum(a, 0.0)
    y = jnp.dot(a, w_ref[...], preferred_element_type=jnp.float32) + b_ref[...]
    y_ref[...] = y

    @pl.when(i == 0)
    def _():
        acc_ref[...] = jnp.zeros_like(acc_ref)

    acc_ref[0:1, :] += jnp.sum(y, axis=0, keepdims=True)
    acc_ref[1:2, :] += jnp.sum(y * y, axis=0, keepdims=True)

    @pl.when(i == nblocks - 1)
    def _():
        m = acc_ref[0:1, :] / total
        st_ref[0:1, :] = m
        st_ref[1:2, :] = acc_ref[1:2, :] / total - m * m


def _final_body(x_ref, st_in_ref, g_ref, bt_ref, o_ref, *, k):
    mu = st_in_ref[0:1, :]
    var = st_in_ref[1:2, :]
    a = g_ref[...] * (x_ref[...] - mu) * jax.lax.rsqrt(var + _EPS) + bt_ref[...]
    a = jnp.maximum(a, 0.0)
    rb = _BM // k
    c = a.shape[-1]
    o_ref[...] = jnp.max(a.reshape(rb, k, c), axis=1)


def _mlp_scale(x_raw, cents, layers, k):
    m = x_raw.shape[0]
    nblocks = m // _BM
    rb = _BM // k
    f32 = jnp.float32

    (w1, b1, g1, t1), (w2, b2, g2, t2), (w3, b3, g3, t3) = layers
    c1, c2, c3 = w1.shape[0], w2.shape[0], w3.shape[0]
    w1t, w2t, w3t = w1.T, w2.T, w3.T

    row_spec = lambda c: pl.BlockSpec((_BM, c), lambda i: (i, 0))
    full = lambda a: pl.BlockSpec(a.shape, lambda i: (0,) * a.ndim)

    y1, st1 = pl.pallas_call(
        functools.partial(_layer1_body, k=k, nblocks=nblocks, total=float(m)),
        grid=(nblocks,),
        in_specs=[row_spec(_CIN), pl.BlockSpec((rb, 3), lambda i: (i, 0)),
                  full(w1t), pl.BlockSpec((1, c1), lambda i: (0, 0))],
        out_specs=[row_spec(c1), pl.BlockSpec((2, c1), lambda i: (0, 0))],
        out_shape=[jax.ShapeDtypeStruct((m, c1), f32),
                   jax.ShapeDtypeStruct((2, c1), f32)],
        scratch_shapes=[pltpu.VMEM((2, c1), f32)],
    )(x_raw, cents, w1t, b1.reshape(1, c1))

    def mid(y, st, g, bt, wt, b, cin, cout):
        return pl.pallas_call(
            functools.partial(_layer_body, nblocks=nblocks, total=float(m)),
            grid=(nblocks,),
            in_specs=[row_spec(cin), pl.BlockSpec((2, cin), lambda i: (0, 0)),
                      pl.BlockSpec((1, cin), lambda i: (0, 0)),
                      pl.BlockSpec((1, cin), lambda i: (0, 0)),
                      full(wt), pl.BlockSpec((1, cout), lambda i: (0, 0))],
            out_specs=[row_spec(cout), pl.BlockSpec((2, cout), lambda i: (0, 0))],
            out_shape=[jax.ShapeDtypeStruct((m, cout), f32),
                       jax.ShapeDtypeStruct((2, cout), f32)],
            scratch_shapes=[pltpu.VMEM((2, cout), f32)],
        )(y, st, g.reshape(1, cin), bt.reshape(1, cin), wt, b.reshape(1, cout))

    y2, st2 = mid(y1, st1, g1, t1, w2t, b2, c1, c2)
    y3, st3 = mid(y2, st2, g2, t2, w3t, b3, c2, c3)

    out = pl.pallas_call(
        functools.partial(_final_body, k=k),
        grid=(nblocks,),
        in_specs=[row_spec(c3), pl.BlockSpec((2, c3), lambda i: (0, 0)),
                  pl.BlockSpec((1, c3), lambda i: (0, 0)),
                  pl.BlockSpec((1, c3), lambda i: (0, 0))],
        out_specs=pl.BlockSpec((rb, c3), lambda i: (i, 0)),
        out_shape=jax.ShapeDtypeStruct((m // k, c3), f32),
    )(y3, st3, g3.reshape(1, c3), t3.reshape(1, c3))
    return out


def _fps_body(x_ref, y_ref, z_ref, cx_ref, cy_ref, cz_ref):
    x = x_ref[...]
    y = y_ref[...]
    z = z_ref[...]
    iota_n = jax.lax.broadcasted_iota(jnp.int32, (_B, _N), 1)
    iota_s = jax.lax.broadcasted_iota(jnp.int32, (_B, _S), 1)
    cx_ref[...] = jnp.zeros_like(cx_ref)
    cy_ref[...] = jnp.zeros_like(cy_ref)
    cz_ref[...] = jnp.zeros_like(cz_ref)

    def body(i, carry):
        dist, far = carry
        oh = (iota_n == far).astype(jnp.float32)
        cx = jnp.sum(x * oh, axis=1, keepdims=True)
        cy = jnp.sum(y * oh, axis=1, keepdims=True)
        cz = jnp.sum(z * oh, axis=1, keepdims=True)
        sel = (iota_s == i).astype(jnp.float32)
        cx_ref[...] += cx * sel
        cy_ref[...] += cy * sel
        cz_ref[...] += cz * sel
        dx = x - cx
        dy = y - cy
        dz = z - cz
        d = dx * dx + dy * dy + dz * dz
        dist = jnp.minimum(dist, d)
        m = jnp.max(dist, axis=1, keepdims=True)
        far2 = jnp.min(jnp.where(dist == m, iota_n, _N), axis=1, keepdims=True)
        return dist, far2.astype(jnp.int32)

    dist0 = jnp.full((_B, _N), 1e10, jnp.float32)
    far0 = jnp.zeros((_B, 1), jnp.int32)
    jax.lax.fori_loop(0, _S, body, (dist0, far0))


def _fps_pallas(points_xyz):
    f32 = jnp.float32
    x = points_xyz[:, :, 0]
    y = points_xyz[:, :, 1]
    z = points_xyz[:, :, 2]
    return pl.pallas_call(
        _fps_body,
        out_shape=[jax.ShapeDtypeStruct((_B, _S), f32)] * 3,
    )(x, y, z)


_NC, _NS = 2, 16
_CSLICE = _S // 4


def _splat(vec, lane):
    dnums = lax.GatherDimensionNumbers(
        offset_dims=(), collapsed_slice_dims=(0,), start_index_map=(0,))
    return lax.gather(vec, lane[:, None], dnums, (1,),
                      mode=lax.GatherScatterMode.PROMISE_IN_BOUNDS)


def _bq_body(px_h, py_h, pz_h, cx_h, cy_h, cz_h, table_h,
             x1_h, x2_h, x3_h,
             pxv, pyv, pzv, cxv, cyv, czv,
             idx1, idx2, idx3, rows1, rows2, rows3, sem):
    wid = lax.axis_index("s") * _NC + lax.axis_index("c")
    b = wid // 4
    sl = wid % 4
    pltpu.sync_copy(px_h.at[b], pxv)
    pltpu.sync_copy(py_h.at[b], pyv)
    pltpu.sync_copy(pz_h.at[b], pzv)
    pltpu.sync_copy(cx_h.at[b, sl], cxv)
    pltpu.sync_copy(cy_h.at[b, sl], cyv)
    pltpu.sync_copy(cz_h.at[b, sl], czv)
    iota = lax.iota(jnp.int32, 16)
    boff = b * _N
    nchunks = _N // 16

    for r, k, idxb, rows, xout in ((_RADII[0], _KS[0], idx1, rows1, x1_h),
                                   (_RADII[1], _KS[1], idx2, rows2, x2_h),
                                   (_RADII[2], _KS[2], idx3, rows3, x3_h)):
        r2 = r * r

        def cent_body(ci, carry, *, r2=r2, k=k, idxb=idxb, rows=rows,
                      xout=xout):
            lane = jnp.full((16,), ci % 16, jnp.int32)
            gbase = (ci // 16) * 16
            cxs = _splat(cxv[pl.ds(gbase, 16)], lane)
            cys = _splat(cyv[pl.ds(gbase, 16)], lane)
            czs = _splat(czv[pl.ds(gbase, 16)], lane)

            def scan_cond(c):
                return (c[0] < nchunks) & (c[1] < k)

            def scan_body(c):
                j, cnt = c
                base = j * 16
                dx = pxv[pl.ds(base, 16)] - cxs
                dy = pyv[pl.ds(base, 16)] - cys
                dz = pzv[pl.ds(base, 16)] - czs
                d = dx * dx + dy * dy + dz * dz
                msk = d <= r2
                cs = plsc.cumsum(msk.astype(jnp.int32))
                pos = cs - 1 + jnp.full((16,), cnt, jnp.int32)
                wm = msk & (pos < k)
                plsc.store_scatter(idxb, [pos], iota + base, mask=wm)
                return j + 1, cnt + jnp.max(cs)

            _, cnt = lax.while_loop(scan_cond, scan_body, (0, 0))

            first = _splat(idxb[pl.ds(0, 16)], jnp.zeros((16,), jnp.int32))
            cntv = jnp.full((16,), cnt, jnp.int32)
            for t in range(k // 16):
                cur = idxb[pl.ds(t * 16, 16)]
                ids = iota + t * 16
                idxb[pl.ds(t * 16, 16)] = (
                    jnp.where(ids >= cntv, first, cur) + boff)
            pltpu.async_copy(table_h.at[idxb], rows, sem).wait()
            base_row = (b * _S + sl * _CSLICE + ci) * k
            pltpu.sync_copy(rows, xout.at[pl.ds(base_row, k)])
            return carry

        lax.fori_loop(0, _CSLICE, cent_body, 0)


def _ballquery_gather_sc(px, py, pz, cx, cy, cz, table_flat):
    f32, i32 = jnp.float32, jnp.int32
    mesh = plsc.VectorSubcoreMesh(core_axis_name="c", subcore_axis_name="s")
    fn = pl.kernel(
        _bq_body,
        out_type=[jax.ShapeDtypeStruct((_B * _S * k, _CIN), f32)
                  for k in _KS],
        compiler_params=pltpu.CompilerParams(
            use_tc_tiling_on_sc=False, needs_layout_passes=False),
        mesh=mesh,
        scratch_types=[
            pltpu.VMEM((_N,), f32), pltpu.VMEM((_N,), f32),
            pltpu.VMEM((_N,), f32),
            pltpu.VMEM((_CSLICE,), f32), pltpu.VMEM((_CSLICE,), f32),
            pltpu.VMEM((_CSLICE,), f32),
            pltpu.VMEM((_KS[0],), i32), pltpu.VMEM((_KS[1],), i32),
            pltpu.VMEM((_KS[2],), i32),
            pltpu.VMEM((_KS[0], _CIN), f32), pltpu.VMEM((_KS[1], _CIN), f32),
            pltpu.VMEM((_KS[2], _CIN), f32),
            pltpu.SemaphoreType.DMA,
        ],
    )
    cx4 = cx.reshape(_B, 4, _CSLICE)
    cy4 = cy.reshape(_B, 4, _CSLICE)
    cz4 = cz.reshape(_B, 4, _CSLICE)
    return fn(px, py, pz, cx4, cy4, cz4, table_flat)


def kernel(points_xyz, features, params):
    cx, cy, cz = _fps_pallas(points_xyz)
    cents = jnp.stack([cx, cy, cz], axis=-1)
    table = jnp.concatenate([points_xyz, features], axis=-1)
    table_flat = table.reshape(_B * _N, _CIN)
    cents_flat = cents.reshape(_B * _S, 3)

    xs = _ballquery_gather_sc(points_xyz[:, :, 0], points_xyz[:, :, 1],
                              points_xyz[:, :, 2], cx, cy, cz, table_flat)
    outs = []
    for x_raw, k, layers in zip(xs, _KS, params):
        out = _mlp_scale(x_raw, cents_flat, layers, k)
        outs.append(out.reshape(_B, _S, -1))
    return cents, jnp.concatenate(outs, axis=-1)


# speedup vs baseline: 20.1887x; 2.0396x over previous
import functools

import jax
import jax.numpy as jnp
from jax import lax
from jax.experimental import pallas as pl
from jax.experimental.pallas import tpu as pltpu
from jax.experimental.pallas import tpu_sc as plsc

_B, _N, _F = 8, 4096, 13
_S = 1024
_RADII = (0.1, 0.2, 0.4)
_KS = (16, 32, 64)
_CIN = 16
_EPS = 1e-5
_BM = 2048


def _layer1_body(x_ref, c_ref, w_ref, b_ref, y_ref, st_ref, acc_ref, *,
                 k, nblocks, total):
    i = pl.program_id(0)
    rb = _BM // k
    x = x_ref[...]
    y = jnp.dot(x, w_ref[...], preferred_element_type=jnp.float32) + b_ref[...]
    corr = jnp.dot(c_ref[...], w_ref[0:3, :], preferred_element_type=jnp.float32)
    cout = y.shape[-1]
    corr = jnp.broadcast_to(corr[:, None, :], (rb, k, cout)).reshape(_BM, cout)
    y = y - corr
    y_ref[...] = y

    @pl.when(i == 0)
    def _():
        acc_ref[...] = jnp.zeros_like(acc_ref)

    acc_ref[0:1, :] += jnp.sum(y, axis=0, keepdims=True)
    acc_ref[1:2, :] += jnp.sum(y * y, axis=0, keepdims=True)

    @pl.when(i == nblocks - 1)
    def _():
        m = acc_ref[0:1, :] / total
        st_ref[0:1, :] = m
        st_ref[1:2, :] = acc_ref[1:2, :] / total - m * m


def _layer_body(x_ref, st_in_ref, g_ref, bt_ref, w_ref, b_ref,
                y_ref, st_ref, acc_ref, *, nblocks, total):
    i = pl.program_id(0)
    mu = st_in_ref[0:1, :]
    var = st_in_ref[1:2, :]
    a = g_ref[...] * (x_ref[...] - mu) * jax.lax.rsqrt(var + _EPS) + bt_ref[...]
    a = jnp.maximum(a, 0.0)
    y = jnp.dot(a, w_ref[...], preferred_element_type=jnp.float32) + b_ref[...]
    y_ref[...] = y

    @pl.when(i == 0)
    def _():
        acc_ref[...] = jnp.zeros_like(acc_ref)

    acc_ref[0:1, :] += jnp.sum(y, axis=0, keepdims=True)
    acc_ref[1:2, :] += jnp.sum(y * y, axis=0, keepdims=True)

    @pl.when(i == nblocks - 1)
    def _():
        m = acc_ref[0:1, :] / total
        st_ref[0:1, :] = m
        st_ref[1:2, :] = acc_ref[1:2, :] / total - m * m


def _final_body(x_ref, st_in_ref, g_ref, bt_ref, o_ref, *, k):
    mu = st_in_ref[0:1, :]
    var = st_in_ref[1:2, :]
    a = g_ref[...] * (x_ref[...] - mu) * jax.lax.rsqrt(var + _EPS) + bt_ref[...]
    a = jnp.maximum(a, 0.0)
    rb = _BM // k
    c = a.shape[-1]
    o_ref[...] = jnp.max(a.reshape(rb, k, c), axis=1)


def _mlp_scale(x_raw, cents, layers, k):
    m = x_raw.shape[0]
    nblocks = m // _BM
    rb = _BM // k
    f32 = jnp.float32

    (w1, b1, g1, t1), (w2, b2, g2, t2), (w3, b3, g3, t3) = layers
    c1, c2, c3 = w1.shape[0], w2.shape[0], w3.shape[0]
    w1t, w2t, w3t = w1.T, w2.T, w3.T

    row_spec = lambda c: pl.BlockSpec((_BM, c), lambda i: (i, 0))
    full = lambda a: pl.BlockSpec(a.shape, lambda i: (0,) * a.ndim)

    y1, st1 = pl.pallas_call(
        functools.partial(_layer1_body, k=k, nblocks=nblocks, total=float(m)),
        grid=(nblocks,),
        in_specs=[row_spec(_CIN), pl.BlockSpec((rb, 3), lambda i: (i, 0)),
                  full(w1t), pl.BlockSpec((1, c1), lambda i: (0, 0))],
        out_specs=[row_spec(c1), pl.BlockSpec((2, c1), lambda i: (0, 0))],
        out_shape=[jax.ShapeDtypeStruct((m, c1), f32),
                   jax.ShapeDtypeStruct((2, c1), f32)],
        scratch_shapes=[pltpu.VMEM((2, c1), f32)],
    )(x_raw, cents, w1t, b1.reshape(1, c1))

    def mid(y, st, g, bt, wt, b, cin, cout):
        return pl.pallas_call(
            functools.partial(_layer_body, nblocks=nblocks, total=float(m)),
            grid=(nblocks,),
            in_specs=[row_spec(cin), pl.BlockSpec((2, cin), lambda i: (0, 0)),
                      pl.BlockSpec((1, cin), lambda i: (0, 0)),
                      pl.BlockSpec((1, cin), lambda i: (0, 0)),
                      full(wt), pl.BlockSpec((1, cout), lambda i: (0, 0))],
            out_specs=[row_spec(cout), pl.BlockSpec((2, cout), lambda i: (0, 0))],
            out_shape=[jax.ShapeDtypeStruct((m, cout), f32),
                       jax.ShapeDtypeStruct((2, cout), f32)],
            scratch_shapes=[pltpu.VMEM((2, cout), f32)],
        )(y, st, g.reshape(1, cin), bt.reshape(1, cin), wt, b.reshape(1, cout))

    y2, st2 = mid(y1, st1, g1, t1, w2t, b2, c1, c2)
    y3, st3 = mid(y2, st2, g2, t2, w3t, b3, c2, c3)

    out = pl.pallas_call(
        functools.partial(_final_body, k=k),
        grid=(nblocks,),
        in_specs=[row_spec(c3), pl.BlockSpec((2, c3), lambda i: (0, 0)),
                  pl.BlockSpec((1, c3), lambda i: (0, 0)),
                  pl.BlockSpec((1, c3), lambda i: (0, 0))],
        out_specs=pl.BlockSpec((rb, c3), lambda i: (i, 0)),
        out_shape=jax.ShapeDtypeStruct((m // k, c3), f32),
    )(y3, st3, g3.reshape(1, c3), t3.reshape(1, c3))
    return out


def _fps_body(x_ref, y_ref, z_ref, cx_ref, cy_ref, cz_ref):
    x = x_ref[...]
    y = y_ref[...]
    z = z_ref[...]
    iota_n = jax.lax.broadcasted_iota(jnp.int32, (_B, _N), 1)
    iota_s = jax.lax.broadcasted_iota(jnp.int32, (_B, _S), 1)
    cx_ref[...] = jnp.zeros_like(cx_ref)
    cy_ref[...] = jnp.zeros_like(cy_ref)
    cz_ref[...] = jnp.zeros_like(cz_ref)

    def body(i, carry):
        dist, far = carry
        oh = (iota_n == far).astype(jnp.float32)
        cx = jnp.sum(x * oh, axis=1, keepdims=True)
        cy = jnp.sum(y * oh, axis=1, keepdims=True)
        cz = jnp.sum(z * oh, axis=1, keepdims=True)
        sel = (iota_s == i).astype(jnp.float32)
        cx_ref[...] += cx * sel
        cy_ref[...] += cy * sel
        cz_ref[...] += cz * sel
        dx = x - cx
        dy = y - cy
        dz = z - cz
        d = dx * dx + dy * dy + dz * dz
        dist = jnp.minimum(dist, d)
        m = jnp.max(dist, axis=1, keepdims=True)
        far2 = jnp.min(jnp.where(dist == m, iota_n, _N), axis=1, keepdims=True)
        return dist, far2.astype(jnp.int32)

    dist0 = jnp.full((_B, _N), 1e10, jnp.float32)
    far0 = jnp.zeros((_B, 1), jnp.int32)
    jax.lax.fori_loop(0, _S, body, (dist0, far0))


def _fps_pallas(points_xyz):
    f32 = jnp.float32
    x = points_xyz[:, :, 0]
    y = points_xyz[:, :, 1]
    z = points_xyz[:, :, 2]
    return pl.pallas_call(
        _fps_body,
        out_shape=[jax.ShapeDtypeStruct((_B, _S), f32)] * 3,
    )(x, y, z)


_NC, _NS = 2, 16
_CSLICE = _S // 4


def _splat(vec, lane):
    dnums = lax.GatherDimensionNumbers(
        offset_dims=(), collapsed_slice_dims=(0,), start_index_map=(0,))
    return lax.gather(vec, lane[:, None], dnums, (1,),
                      mode=lax.GatherScatterMode.PROMISE_IN_BOUNDS)


def _bq_body(px_h, py_h, pz_h, cx_h, cy_h, cz_h, table_h, x_h,
             pxv, pyv, pzv, cxv, cyv, czv, idxb, rows, sem, *, r, k):
    wid = lax.axis_index("s") * _NC + lax.axis_index("c")
    b = wid // 4
    sl = wid % 4
    pltpu.sync_copy(px_h.at[b], pxv)
    pltpu.sync_copy(py_h.at[b], pyv)
    pltpu.sync_copy(pz_h.at[b], pzv)
    pltpu.sync_copy(cx_h.at[b, sl], cxv)
    pltpu.sync_copy(cy_h.at[b, sl], cyv)
    pltpu.sync_copy(cz_h.at[b, sl], czv)
    iota = lax.iota(jnp.int32, 16)
    boff = b * _N
    ngroups = _N // 64
    r2 = r * r

    def cent_body(ci, carry):
        lane = jnp.full((16,), ci % 16, jnp.int32)
        gbase = (ci // 16) * 16
        cxs = _splat(cxv[pl.ds(gbase, 16)], lane)
        cys = _splat(cyv[pl.ds(gbase, 16)], lane)
        czs = _splat(czv[pl.ds(gbase, 16)], lane)

        def scan_cond(c):
            return (c[0] < ngroups) & (c[1] < k)

        def scan_body(c):
            g, cnt = c
            base = g * 64
            masks = []
            for t in range(4):
                dx = pxv[pl.ds(base + t * 16, 16)] - cxs
                dy = pyv[pl.ds(base + t * 16, 16)] - cys
                dz = pzv[pl.ds(base + t * 16, 16)] - czs
                masks.append(dx * dx + dy * dy + dz * dz <= r2)
            anyv = (masks[0] | masks[1]) | (masks[2] | masks[3])

            def do_sel():
                off = jnp.full((16,), cnt, jnp.int32)
                for t in range(4):
                    cs = plsc.cumsum(masks[t].astype(jnp.int32))
                    pos = cs - 1 + off
                    wm = masks[t] & (pos < k)
                    plsc.store_scatter(idxb, [pos], iota + base + t * 16,
                                       mask=wm)
                    off = off + plsc.all_reduce_population_count(masks[t])
                return jnp.max(off)

            cnt2 = lax.cond(jnp.any(anyv), do_sel, lambda: cnt)
            return g + 1, cnt2

        _, cnt = lax.while_loop(scan_cond, scan_body, (0, 0))

        first = _splat(idxb[pl.ds(0, 16)], jnp.zeros((16,), jnp.int32))
        cntv = jnp.full((16,), cnt, jnp.int32)
        for t in range(k // 16):
            cur = idxb[pl.ds(t * 16, 16)]
            ids = iota + t * 16
            idxb[pl.ds(t * 16, 16)] = jnp.where(ids >= cntv, first, cur) + boff
        pltpu.async_copy(table_h.at[idxb], rows, sem).wait()
        base_row = (b * _S + sl * _CSLICE + ci) * k
        pltpu.sync_copy(rows, x_h.at[pl.ds(base_row, k)])
        return carry

    lax.fori_loop(0, _CSLICE, cent_body, 0)


def _ballquery_gather_sc(px, py, pz, cx, cy, cz, table_flat):
    f32, i32 = jnp.float32, jnp.int32
    cx4 = cx.reshape(_B, 4, _CSLICE)
    cy4 = cy.reshape(_B, 4, _CSLICE)
    cz4 = cz.reshape(_B, 4, _CSLICE)
    outs = []
    for r, k in zip(_RADII, _KS):
        mesh = plsc.VectorSubcoreMesh(core_axis_name="c",
                                      subcore_axis_name="s")
        fn = pl.kernel(
            functools.partial(_bq_body, r=r, k=k),
            out_type=jax.ShapeDtypeStruct((_B * _S * k, _CIN), f32),
            compiler_params=pltpu.CompilerParams(
                use_tc_tiling_on_sc=False, needs_layout_passes=False),
            mesh=mesh,
            scratch_types=[
                pltpu.VMEM((_N,), f32), pltpu.VMEM((_N,), f32),
                pltpu.VMEM((_N,), f32),
                pltpu.VMEM((_CSLICE,), f32), pltpu.VMEM((_CSLICE,), f32),
                pltpu.VMEM((_CSLICE,), f32),
                pltpu.VMEM((k,), i32),
                pltpu.VMEM((k, _CIN), f32),
                pltpu.SemaphoreType.DMA,
            ],
        )
        outs.append(fn(px, py, pz, cx4, cy4, cz4, table_flat))
    return outs


def kernel(points_xyz, features, params):
    cx, cy, cz = _fps_pallas(points_xyz)
    cents = jnp.stack([cx, cy, cz], axis=-1)
    table = jnp.concatenate([points_xyz, features], axis=-1)
    table_flat = table.reshape(_B * _N, _CIN)
    cents_flat = cents.reshape(_B * _S, 3)

    xs = _ballquery_gather_sc(points_xyz[:, :, 0], points_xyz[:, :, 1],
                              points_xyz[:, :, 2], cx, cy, cz, table_flat)
    outs = []
    for x_raw, k, layers in zip(xs, _KS, params):
        out = _mlp_scale(x_raw, cents_flat, layers, k)
        outs.append(out.reshape(_B, _S, -1))
    return cents, jnp.concatenate(outs, axis=-1)


# speedup vs baseline: 21.1644x; 1.0483x over previous
import functools

import jax
import jax.numpy as jnp
from jax import lax
from jax.experimental import pallas as pl
from jax.experimental.pallas import tpu as pltpu
from jax.experimental.pallas import tpu_sc as plsc

_B, _N, _F = 8, 4096, 13
_S = 1024
_RADII = (0.1, 0.2, 0.4)
_KS = (16, 32, 64)
_CIN = 16
_EPS = 1e-5
_BM = 2048


def _layer1_body(x_ref, c_ref, w_ref, b_ref, y_ref, st_ref, acc_ref, *,
                 k, nblocks, total):
    i = pl.program_id(0)
    rb = _BM // k
    x = x_ref[...]
    y = jnp.dot(x, w_ref[...], preferred_element_type=jnp.float32) + b_ref[...]
    corr = jnp.dot(c_ref[...], w_ref[0:3, :], preferred_element_type=jnp.float32)
    cout = y.shape[-1]
    corr = jnp.broadcast_to(corr[:, None, :], (rb, k, cout)).reshape(_BM, cout)
    y = y - corr
    y_ref[...] = y

    @pl.when(i == 0)
    def _():
        acc_ref[...] = jnp.zeros_like(acc_ref)

    acc_ref[0:1, :] += jnp.sum(y, axis=0, keepdims=True)
    acc_ref[1:2, :] += jnp.sum(y * y, axis=0, keepdims=True)

    @pl.when(i == nblocks - 1)
    def _():
        m = acc_ref[0:1, :] / total
        st_ref[0:1, :] = m
        st_ref[1:2, :] = acc_ref[1:2, :] / total - m * m


def _layer_body(x_ref, st_in_ref, g_ref, bt_ref, w_ref, b_ref,
                y_ref, st_ref, acc_ref, *, nblocks, total):
    i = pl.program_id(0)
    mu = st_in_ref[0:1, :]
    var = st_in_ref[1:2, :]
    a = g_ref[...] * (x_ref[...] - mu) * jax.lax.rsqrt(var + _EPS) + bt_ref[...]
    a = jnp.maximum(a, 0.0)
    y = jnp.dot(a, w_ref[...], preferred_element_type=jnp.float32) + b_ref[...]
    y_ref[...] = y

    @pl.when(i == 0)
    def _():
        acc_ref[...] = jnp.zeros_like(acc_ref)

    acc_ref[0:1, :] += jnp.sum(y, axis=0, keepdims=True)
    acc_ref[1:2, :] += jnp.sum(y * y, axis=0, keepdims=True)

    @pl.when(i == nblocks - 1)
    def _():
        m = acc_ref[0:1, :] / total
        st_ref[0:1, :] = m
        st_ref[1:2, :] = acc_ref[1:2, :] / total - m * m


def _final_body(x_ref, st_in_ref, g_ref, bt_ref, o_ref, *, k):
    mu = st_in_ref[0:1, :]
    var = st_in_ref[1:2, :]
    a = g_ref[...] * (x_ref[...] - mu) * jax.lax.rsqrt(var + _EPS) + bt_ref[...]
    a = jnp.maximum(a, 0.0)
    rb = _BM // k
    c = a.shape[-1]
    o_ref[...] = jnp.max(a.reshape(rb, k, c), axis=1)


def _mlp_scale(x_raw, cents, layers, k):
    m = x_raw.shape[0]
    nblocks = m // _BM
    rb = _BM // k
    f32 = jnp.float32

    (w1, b1, g1, t1), (w2, b2, g2, t2), (w3, b3, g3, t3) = layers
    c1, c2, c3 = w1.shape[0], w2.shape[0], w3.shape[0]
    w1t, w2t, w3t = w1.T, w2.T, w3.T

    row_spec = lambda c: pl.BlockSpec((_BM, c), lambda i: (i, 0))
    full = lambda a: pl.BlockSpec(a.shape, lambda i: (0,) * a.ndim)

    y1, st1 = pl.pallas_call(
        functools.partial(_layer1_body, k=k, nblocks=nblocks, total=float(m)),
        grid=(nblocks,),
        in_specs=[row_spec(_CIN), pl.BlockSpec((rb, 3), lambda i: (i, 0)),
                  full(w1t), pl.BlockSpec((1, c1), lambda i: (0, 0))],
        out_specs=[row_spec(c1), pl.BlockSpec((2, c1), lambda i: (0, 0))],
        out_shape=[jax.ShapeDtypeStruct((m, c1), f32),
                   jax.ShapeDtypeStruct((2, c1), f32)],
        scratch_shapes=[pltpu.VMEM((2, c1), f32)],
    )(x_raw, cents, w1t, b1.reshape(1, c1))

    def mid(y, st, g, bt, wt, b, cin, cout):
        return pl.pallas_call(
            functools.partial(_layer_body, nblocks=nblocks, total=float(m)),
            grid=(nblocks,),
            in_specs=[row_spec(cin), pl.BlockSpec((2, cin), lambda i: (0, 0)),
                      pl.BlockSpec((1, cin), lambda i: (0, 0)),
                      pl.BlockSpec((1, cin), lambda i: (0, 0)),
                      full(wt), pl.BlockSpec((1, cout), lambda i: (0, 0))],
            out_specs=[row_spec(cout), pl.BlockSpec((2, cout), lambda i: (0, 0))],
            out_shape=[jax.ShapeDtypeStruct((m, cout), f32),
                       jax.ShapeDtypeStruct((2, cout), f32)],
            scratch_shapes=[pltpu.VMEM((2, cout), f32)],
        )(y, st, g.reshape(1, cin), bt.reshape(1, cin), wt, b.reshape(1, cout))

    y2, st2 = mid(y1, st1, g1, t1, w2t, b2, c1, c2)
    y3, st3 = mid(y2, st2, g2, t2, w3t, b3, c2, c3)

    out = pl.pallas_call(
        functools.partial(_final_body, k=k),
        grid=(nblocks,),
        in_specs=[row_spec(c3), pl.BlockSpec((2, c3), lambda i: (0, 0)),
                  pl.BlockSpec((1, c3), lambda i: (0, 0)),
                  pl.BlockSpec((1, c3), lambda i: (0, 0))],
        out_specs=pl.BlockSpec((rb, c3), lambda i: (i, 0)),
        out_shape=jax.ShapeDtypeStruct((m // k, c3), f32),
    )(y3, st3, g3.reshape(1, c3), t3.reshape(1, c3))
    return out


def _fps_body(x_ref, y_ref, z_ref, cx_ref, cy_ref, cz_ref):
    x = x_ref[...]
    y = y_ref[...]
    z = z_ref[...]
    iota_n = jax.lax.broadcasted_iota(jnp.int32, (_B, _N), 1)
    iota_s = jax.lax.broadcasted_iota(jnp.int32, (_B, _S), 1)
    cx_ref[...] = jnp.zeros_like(cx_ref)
    cy_ref[...] = jnp.zeros_like(cy_ref)
    cz_ref[...] = jnp.zeros_like(cz_ref)

    def body(i, carry):
        dist, far = carry
        oh = (iota_n == far).astype(jnp.float32)
        cx = jnp.sum(x * oh, axis=1, keepdims=True)
        cy = jnp.sum(y * oh, axis=1, keepdims=True)
        cz = jnp.sum(z * oh, axis=1, keepdims=True)
        sel = (iota_s == i).astype(jnp.float32)
        cx_ref[...] += cx * sel
        cy_ref[...] += cy * sel
        cz_ref[...] += cz * sel
        dx = x - cx
        dy = y - cy
        dz = z - cz
        d = dx * dx + dy * dy + dz * dz
        dist = jnp.minimum(dist, d)
        m = jnp.max(dist, axis=1, keepdims=True)
        far2 = jnp.min(jnp.where(dist == m, iota_n, _N), axis=1, keepdims=True)
        return dist, far2.astype(jnp.int32)

    dist0 = jnp.full((_B, _N), 1e10, jnp.float32)
    far0 = jnp.zeros((_B, 1), jnp.int32)
    jax.lax.fori_loop(0, _S, body, (dist0, far0))


def _fps_pallas(points_xyz):
    f32 = jnp.float32
    x = points_xyz[:, :, 0]
    y = points_xyz[:, :, 1]
    z = points_xyz[:, :, 2]
    return pl.pallas_call(
        _fps_body,
        out_shape=[jax.ShapeDtypeStruct((_B, _S), f32)] * 3,
    )(x, y, z)


_NC, _NS = 2, 16
_CSLICE = _S // 4


def _splat(vec, lane):
    dnums = lax.GatherDimensionNumbers(
        offset_dims=(), collapsed_slice_dims=(0,), start_index_map=(0,))
    return lax.gather(vec, lane[:, None], dnums, (1,),
                      mode=lax.GatherScatterMode.PROMISE_IN_BOUNDS)


def _bq_body(px_h, py_h, pz_h, cx_h, cy_h, cz_h, table_h, x_h,
             pxv, pyv, pzv, cxv, cyv, czv,
             idx0, idx1, rows0, rows1, sem0, sem1, *, r, k):
    wid = lax.axis_index("s") * _NC + lax.axis_index("c")
    b = wid // 4
    sl = wid % 4
    pltpu.sync_copy(px_h.at[b], pxv)
    pltpu.sync_copy(py_h.at[b], pyv)
    pltpu.sync_copy(pz_h.at[b], pzv)
    pltpu.sync_copy(cx_h.at[b, sl], cxv)
    pltpu.sync_copy(cy_h.at[b, sl], cyv)
    pltpu.sync_copy(cz_h.at[b, sl], czv)
    iota = lax.iota(jnp.int32, 16)
    boff = b * _N
    ngroups = _N // 64
    r2 = r * r
    idxbs, rowss, sems = (idx0, idx1), (rows0, rows1), (sem0, sem1)

    def scan_pad(ci, idxb):
        lane = jnp.full((16,), ci % 16, jnp.int32)
        gbase = (ci // 16) * 16
        cxs = _splat(cxv[pl.ds(gbase, 16)], lane)
        cys = _splat(cyv[pl.ds(gbase, 16)], lane)
        czs = _splat(czv[pl.ds(gbase, 16)], lane)

        def scan_cond(c):
            return (c[0] < ngroups) & (c[1] < k)

        def scan_body(c):
            g, cnt = c
            base = g * 64
            masks = []
            for t in range(4):
                dx = pxv[pl.ds(base + t * 16, 16)] - cxs
                dy = pyv[pl.ds(base + t * 16, 16)] - cys
                dz = pzv[pl.ds(base + t * 16, 16)] - czs
                masks.append(dx * dx + dy * dy + dz * dz <= r2)
            anyv = (masks[0] | masks[1]) | (masks[2] | masks[3])

            def do_sel():
                off = jnp.full((16,), cnt, jnp.int32)
                for t in range(4):
                    cs = plsc.cumsum(masks[t].astype(jnp.int32))
                    pos = cs - 1 + off
                    wm = masks[t] & (pos < k)
                    plsc.store_scatter(idxb, [pos], iota + base + t * 16,
                                       mask=wm)
                    off = off + plsc.all_reduce_population_count(masks[t])
                return jnp.max(off)

            cnt2 = lax.cond(jnp.any(anyv), do_sel, lambda: cnt)
            return g + 1, cnt2

        _, cnt = lax.while_loop(scan_cond, scan_body, (0, 0))

        first = _splat(idxb[pl.ds(0, 16)], jnp.zeros((16,), jnp.int32))
        cntv = jnp.full((16,), cnt, jnp.int32)
        for t in range(k // 16):
            cur = idxb[pl.ds(t * 16, 16)]
            ids = iota + t * 16
            idxb[pl.ds(t * 16, 16)] = jnp.where(ids >= cntv, first, cur) + boff

    def out_base(c):
        return (b * _S + sl * _CSLICE + c) * k

    for s in (0, 1):
        scan_pad(s, idxbs[s])
        pltpu.async_copy(table_h.at[idxbs[s]], rowss[s], sems[s])

    def pair_body(p, carry):
        for s in (0, 1):
            c = 2 * p + s
            pltpu.make_async_copy(table_h.at[idxbs[s]], rowss[s],
                                  sems[s]).wait()
            pltpu.sync_copy(rowss[s], x_h.at[pl.ds(out_base(c - 2), k)])
            scan_pad(c, idxbs[s])
            pltpu.async_copy(table_h.at[idxbs[s]], rowss[s], sems[s])
        return carry

    lax.fori_loop(1, _CSLICE // 2, pair_body, 0)

    for s in (0, 1):
        pltpu.make_async_copy(table_h.at[idxbs[s]], rowss[s], sems[s]).wait()
        pltpu.sync_copy(rowss[s], x_h.at[pl.ds(out_base(_CSLICE - 2 + s), k)])


def _ballquery_gather_sc(px, py, pz, cx, cy, cz, table_flat):
    f32, i32 = jnp.float32, jnp.int32
    cx4 = cx.reshape(_B, 4, _CSLICE)
    cy4 = cy.reshape(_B, 4, _CSLICE)
    cz4 = cz.reshape(_B, 4, _CSLICE)
    outs = []
    for r, k in zip(_RADII, _KS):
        mesh = plsc.VectorSubcoreMesh(core_axis_name="c",
                                      subcore_axis_name="s")
        fn = pl.kernel(
            functools.partial(_bq_body, r=r, k=k),
            out_type=jax.ShapeDtypeStruct((_B * _S * k, _CIN), f32),
            compiler_params=pltpu.CompilerParams(
                use_tc_tiling_on_sc=False, needs_layout_passes=False),
            mesh=mesh,
            scratch_types=[
                pltpu.VMEM((_N,), f32), pltpu.VMEM((_N,), f32),
                pltpu.VMEM((_N,), f32),
                pltpu.VMEM((_CSLICE,), f32), pltpu.VMEM((_CSLICE,), f32),
                pltpu.VMEM((_CSLICE,), f32),
                pltpu.VMEM((k,), i32), pltpu.VMEM((k,), i32),
                pltpu.VMEM((k, _CIN), f32), pltpu.VMEM((k, _CIN), f32),
                pltpu.SemaphoreType.DMA, pltpu.SemaphoreType.DMA,
            ],
        )
        outs.append(fn(px, py, pz, cx4, cy4, cz4, table_flat))
    return outs


def kernel(points_xyz, features, params):
    cx, cy, cz = _fps_pallas(points_xyz)
    cents = jnp.stack([cx, cy, cz], axis=-1)
    table = jnp.concatenate([points_xyz, features], axis=-1)
    table_flat = table.reshape(_B * _N, _CIN)
    cents_flat = cents.reshape(_B * _S, 3)

    xs = _ballquery_gather_sc(points_xyz[:, :, 0], points_xyz[:, :, 1],
                              points_xyz[:, :, 2], cx, cy, cz, table_flat)
    outs = []
    for x_raw, k, layers in zip(xs, _KS, params):
        out = _mlp_scale(x_raw, cents_flat, layers, k)
        outs.append(out.reshape(_B, _S, -1))
    return cents, jnp.concatenate(outs, axis=-1)


# speedup vs baseline: 21.6476x; 1.0228x over previous
import functools

import jax
import jax.numpy as jnp
from jax import lax
from jax.experimental import pallas as pl
from jax.experimental.pallas import tpu as pltpu
from jax.experimental.pallas import tpu_sc as plsc

_B, _N, _F = 8, 4096, 13
_S = 1024
_RADII = (0.1, 0.2, 0.4)
_KS = (16, 32, 64)
_CIN = 16
_EPS = 1e-5
_BM = 2048


def _layer1_body(x_ref, c_ref, w_ref, b_ref, y_ref, st_ref, acc_ref, *,
                 k, nblocks, total):
    i = pl.program_id(0)
    rb = _BM // k
    x = x_ref[...]
    y = jnp.dot(x, w_ref[...], preferred_element_type=jnp.float32) + b_ref[...]
    corr = jnp.dot(c_ref[...], w_ref[0:3, :], preferred_element_type=jnp.float32)
    cout = y.shape[-1]
    corr = jnp.broadcast_to(corr[:, None, :], (rb, k, cout)).reshape(_BM, cout)
    y = y - corr
    y_ref[...] = y

    @pl.when(i == 0)
    def _():
        acc_ref[...] = jnp.zeros_like(acc_ref)

    acc_ref[0:1, :] += jnp.sum(y, axis=0, keepdims=True)
    acc_ref[1:2, :] += jnp.sum(y * y, axis=0, keepdims=True)

    @pl.when(i == nblocks - 1)
    def _():
        m = acc_ref[0:1, :] / total
        st_ref[0:1, :] = m
        st_ref[1:2, :] = acc_ref[1:2, :] / total - m * m


def _layer_body(x_ref, st_in_ref, g_ref, bt_ref, w_ref, b_ref,
                y_ref, st_ref, acc_ref, *, nblocks, total):
    i = pl.program_id(0)
    mu = st_in_ref[0:1, :]
    var = st_in_ref[1:2, :]
    a = g_ref[...] * (x_ref[...] - mu) * jax.lax.rsqrt(var + _EPS) + bt_ref[...]
    a = jnp.maximum(a, 0.0)
    y = jnp.dot(a, w_ref[...], preferred_element_type=jnp.float32) + b_ref[...]
    y_ref[...] = y

    @pl.when(i == 0)
    def _():
        acc_ref[...] = jnp.zeros_like(acc_ref)

    acc_ref[0:1, :] += jnp.sum(y, axis=0, keepdims=True)
    acc_ref[1:2, :] += jnp.sum(y * y, axis=0, keepdims=True)

    @pl.when(i == nblocks - 1)
    def _():
        m = acc_ref[0:1, :] / total
        st_ref[0:1, :] = m
        st_ref[1:2, :] = acc_ref[1:2, :] / total - m * m


def _stats_body(x_ref, st_in_ref, g_ref, bt_ref, w_ref, b_ref,
                st_ref, acc_ref, *, nblocks, total):
    i = pl.program_id(0)
    mu = st_in_ref[0:1, :]
    var = st_in_ref[1:2, :]
    a = g_ref[...] * (x_ref[...] - mu) * jax.lax.rsqrt(var + _EPS) + bt_ref[...]
    a = jnp.maximum(a, 0.0)
    y = jnp.dot(a, w_ref[...], preferred_element_type=jnp.float32) + b_ref[...]

    @pl.when(i == 0)
    def _():
        acc_ref[...] = jnp.zeros_like(acc_ref)

    acc_ref[0:1, :] += jnp.sum(y, axis=0, keepdims=True)
    acc_ref[1:2, :] += jnp.sum(y * y, axis=0, keepdims=True)

    @pl.when(i == nblocks - 1)
    def _():
        m = acc_ref[0:1, :] / total
        st_ref[0:1, :] = m
        st_ref[1:2, :] = acc_ref[1:2, :] / total - m * m


def _final_body(x_ref, st2_ref, g2_ref, bt2_ref, w_ref, b_ref,
                st3_ref, g3_ref, bt3_ref, o_ref, *, k):
    mu2 = st2_ref[0:1, :]
    var2 = st2_ref[1:2, :]
    a = (g2_ref[...] * (x_ref[...] - mu2) * jax.lax.rsqrt(var2 + _EPS)
         + bt2_ref[...])
    a = jnp.maximum(a, 0.0)
    y = jnp.dot(a, w_ref[...], preferred_element_type=jnp.float32) + b_ref[...]
    mu3 = st3_ref[0:1, :]
    var3 = st3_ref[1:2, :]
    a3 = (g3_ref[...] * (y - mu3) * jax.lax.rsqrt(var3 + _EPS) + bt3_ref[...])
    a3 = jnp.maximum(a3, 0.0)
    rb = _BM // k
    c = a3.shape[-1]
    o_ref[...] = jnp.max(a3.reshape(rb, k, c), axis=1)


def _mlp_scale(x_raw, cents, layers, k):
    m = x_raw.shape[0]
    nblocks = m // _BM
    rb = _BM // k
    f32 = jnp.float32

    (w1, b1, g1, t1), (w2, b2, g2, t2), (w3, b3, g3, t3) = layers
    c1, c2, c3 = w1.shape[0], w2.shape[0], w3.shape[0]
    w1t, w2t, w3t = w1.T, w2.T, w3.T

    row_spec = lambda c: pl.BlockSpec((_BM, c), lambda i: (i, 0))
    full = lambda a: pl.BlockSpec(a.shape, lambda i: (0,) * a.ndim)

    y1, st1 = pl.pallas_call(
        functools.partial(_layer1_body, k=k, nblocks=nblocks, total=float(m)),
        grid=(nblocks,),
        in_specs=[row_spec(_CIN), pl.BlockSpec((rb, 3), lambda i: (i, 0)),
                  full(w1t), pl.BlockSpec((1, c1), lambda i: (0, 0))],
        out_specs=[row_spec(c1), pl.BlockSpec((2, c1), lambda i: (0, 0))],
        out_shape=[jax.ShapeDtypeStruct((m, c1), f32),
                   jax.ShapeDtypeStruct((2, c1), f32)],
        scratch_shapes=[pltpu.VMEM((2, c1), f32)],
    )(x_raw, cents, w1t, b1.reshape(1, c1))

    def mid(y, st, g, bt, wt, b, cin, cout):
        return pl.pallas_call(
            functools.partial(_layer_body, nblocks=nblocks, total=float(m)),
            grid=(nblocks,),
            in_specs=[row_spec(cin), pl.BlockSpec((2, cin), lambda i: (0, 0)),
                      pl.BlockSpec((1, cin), lambda i: (0, 0)),
                      pl.BlockSpec((1, cin), lambda i: (0, 0)),
                      full(wt), pl.BlockSpec((1, cout), lambda i: (0, 0))],
            out_specs=[row_spec(cout), pl.BlockSpec((2, cout), lambda i: (0, 0))],
            out_shape=[jax.ShapeDtypeStruct((m, cout), f32),
                       jax.ShapeDtypeStruct((2, cout), f32)],
            scratch_shapes=[pltpu.VMEM((2, cout), f32)],
        )(y, st, g.reshape(1, cin), bt.reshape(1, cin), wt, b.reshape(1, cout))

    y2, st2 = mid(y1, st1, g1, t1, w2t, b2, c1, c2)

    st3 = pl.pallas_call(
        functools.partial(_stats_body, nblocks=nblocks, total=float(m)),
        grid=(nblocks,),
        in_specs=[row_spec(c2), pl.BlockSpec((2, c2), lambda i: (0, 0)),
                  pl.BlockSpec((1, c2), lambda i: (0, 0)),
                  pl.BlockSpec((1, c2), lambda i: (0, 0)),
                  full(w3t), pl.BlockSpec((1, c3), lambda i: (0, 0))],
        out_specs=pl.BlockSpec((2, c3), lambda i: (0, 0)),
        out_shape=jax.ShapeDtypeStruct((2, c3), f32),
        scratch_shapes=[pltpu.VMEM((2, c3), f32)],
    )(y2, st2, g2.reshape(1, c2), t2.reshape(1, c2), w3t, b3.reshape(1, c3))

    out = pl.pallas_call(
        functools.partial(_final_body, k=k),
        grid=(nblocks,),
        in_specs=[row_spec(c2), pl.BlockSpec((2, c2), lambda i: (0, 0)),
                  pl.BlockSpec((1, c2), lambda i: (0, 0)),
                  pl.BlockSpec((1, c2), lambda i: (0, 0)),
                  full(w3t), pl.BlockSpec((1, c3), lambda i: (0, 0)),
                  pl.BlockSpec((2, c3), lambda i: (0, 0)),
                  pl.BlockSpec((1, c3), lambda i: (0, 0)),
                  pl.BlockSpec((1, c3), lambda i: (0, 0))],
        out_specs=pl.BlockSpec((rb, c3), lambda i: (i, 0)),
        out_shape=jax.ShapeDtypeStruct((m // k, c3), f32),
    )(y2, st2, g2.reshape(1, c2), t2.reshape(1, c2), w3t, b3.reshape(1, c3),
      st3, g3.reshape(1, c3), t3.reshape(1, c3))
    return out


def _fps_body(x_ref, y_ref, z_ref, cx_ref, cy_ref, cz_ref):
    x = x_ref[...]
    y = y_ref[...]
    z = z_ref[...]
    iota_n = jax.lax.broadcasted_iota(jnp.int32, (_B, _N), 1)
    iota_s = jax.lax.broadcasted_iota(jnp.int32, (_B, _S), 1)
    cx_ref[...] = jnp.zeros_like(cx_ref)
    cy_ref[...] = jnp.zeros_like(cy_ref)
    cz_ref[...] = jnp.zeros_like(cz_ref)

    def body(i, carry):
        dist, far = carry
        oh = (iota_n == far).astype(jnp.float32)
        cx = jnp.sum(x * oh, axis=1, keepdims=True)
        cy = jnp.sum(y * oh, axis=1, keepdims=True)
        cz = jnp.sum(z * oh, axis=1, keepdims=True)
        sel = (iota_s == i).astype(jnp.float32)
        cx_ref[...] += cx * sel
        cy_ref[...] += cy * sel
        cz_ref[...] += cz * sel
        dx = x - cx
        dy = y - cy
        dz = z - cz
        d = dx * dx + dy * dy + dz * dz
        dist = jnp.minimum(dist, d)
        m = jnp.max(dist, axis=1, keepdims=True)
        far2 = jnp.min(jnp.where(dist == m, iota_n, _N), axis=1, keepdims=True)
        return dist, far2.astype(jnp.int32)

    dist0 = jnp.full((_B, _N), 1e10, jnp.float32)
    far0 = jnp.zeros((_B, 1), jnp.int32)
    jax.lax.fori_loop(0, _S, body, (dist0, far0))


def _fps_pallas(points_xyz):
    f32 = jnp.float32
    x = points_xyz[:, :, 0]
    y = points_xyz[:, :, 1]
    z = points_xyz[:, :, 2]
    return pl.pallas_call(
        _fps_body,
        out_shape=[jax.ShapeDtypeStruct((_B, _S), f32)] * 3,
    )(x, y, z)


_NC, _NS = 2, 16
_CSLICE = _S // 4


def _splat(vec, lane):
    dnums = lax.GatherDimensionNumbers(
        offset_dims=(), collapsed_slice_dims=(0,), start_index_map=(0,))
    return lax.gather(vec, lane[:, None], dnums, (1,),
                      mode=lax.GatherScatterMode.PROMISE_IN_BOUNDS)


def _bq_body(px_h, py_h, pz_h, cx_h, cy_h, cz_h, table_h, x_h,
             pxv, pyv, pzv, cxv, cyv, czv,
             idx0, idx1, rows0, rows1, sem0, sem1, *, r, k):
    wid = lax.axis_index("s") * _NC + lax.axis_index("c")
    b = wid // 4
    sl = wid % 4
    pltpu.sync_copy(px_h.at[b], pxv)
    pltpu.sync_copy(py_h.at[b], pyv)
    pltpu.sync_copy(pz_h.at[b], pzv)
    pltpu.sync_copy(cx_h.at[b, sl], cxv)
    pltpu.sync_copy(cy_h.at[b, sl], cyv)
    pltpu.sync_copy(cz_h.at[b, sl], czv)
    iota = lax.iota(jnp.int32, 16)
    boff = b * _N
    ngroups = _N // 64
    r2 = r * r
    idxbs, rowss, sems = (idx0, idx1), (rows0, rows1), (sem0, sem1)

    def scan_pad(ci, idxb):
        lane = jnp.full((16,), ci % 16, jnp.int32)
        gbase = (ci // 16) * 16
        cxs = _splat(cxv[pl.ds(gbase, 16)], lane)
        cys = _splat(cyv[pl.ds(gbase, 16)], lane)
        czs = _splat(czv[pl.ds(gbase, 16)], lane)

        def scan_cond(c):
            return (c[0] < ngroups) & (c[1] < k)

        def scan_body(c):
            g, cnt = c
            base = g * 64
            masks = []
            for t in range(4):
                dx = pxv[pl.ds(base + t * 16, 16)] - cxs
                dy = pyv[pl.ds(base + t * 16, 16)] - cys
                dz = pzv[pl.ds(base + t * 16, 16)] - czs
                masks.append(dx * dx + dy * dy + dz * dz <= r2)
            anyv = (masks[0] | masks[1]) | (masks[2] | masks[3])

            def do_sel():
                off = jnp.full((16,), cnt, jnp.int32)
                for t in range(4):
                    cs = plsc.cumsum(masks[t].astype(jnp.int32))
                    pos = cs - 1 + off
                    wm = masks[t] & (pos < k)
                    plsc.store_scatter(idxb, [pos], iota + base + t * 16,
                                       mask=wm)
                    off = off + plsc.all_reduce_population_count(masks[t])
                return jnp.max(off)

            cnt2 = lax.cond(jnp.any(anyv), do_sel, lambda: cnt)
            return g + 1, cnt2

        _, cnt = lax.while_loop(scan_cond, scan_body, (0, 0))

        first = _splat(idxb[pl.ds(0, 16)], jnp.zeros((16,), jnp.int32))
        cntv = jnp.full((16,), cnt, jnp.int32)
        for t in range(k // 16):
            cur = idxb[pl.ds(t * 16, 16)]
            ids = iota + t * 16
            idxb[pl.ds(t * 16, 16)] = jnp.where(ids >= cntv, first, cur) + boff

    def out_base(c):
        return (b * _S + sl * _CSLICE + c) * k

    for s in (0, 1):
        scan_pad(s, idxbs[s])
        pltpu.async_copy(table_h.at[idxbs[s]], rowss[s], sems[s])

    def pair_body(p, carry):
        for s in (0, 1):
            c = 2 * p + s
            pltpu.make_async_copy(table_h.at[idxbs[s]], rowss[s],
                                  sems[s]).wait()
            pltpu.sync_copy(rowss[s], x_h.at[pl.ds(out_base(c - 2), k)])
            scan_pad(c, idxbs[s])
            pltpu.async_copy(table_h.at[idxbs[s]], rowss[s], sems[s])
        return carry

    lax.fori_loop(1, _CSLICE // 2, pair_body, 0)

    for s in (0, 1):
        pltpu.make_async_copy(table_h.at[idxbs[s]], rowss[s], sems[s]).wait()
        pltpu.sync_copy(rowss[s], x_h.at[pl.ds(out_base(_CSLICE - 2 + s), k)])


def _ballquery_gather_sc(px, py, pz, cx, cy, cz, table_flat):
    f32, i32 = jnp.float32, jnp.int32
    cx4 = cx.reshape(_B, 4, _CSLICE)
    cy4 = cy.reshape(_B, 4, _CSLICE)
    cz4 = cz.reshape(_B, 4, _CSLICE)
    outs = []
    for r, k in zip(_RADII, _KS):
        mesh = plsc.VectorSubcoreMesh(core_axis_name="c",
                                      subcore_axis_name="s")
        fn = pl.kernel(
            functools.partial(_bq_body, r=r, k=k),
            out_type=jax.ShapeDtypeStruct((_B * _S * k, _CIN), f32),
            compiler_params=pltpu.CompilerParams(
                use_tc_tiling_on_sc=False, needs_layout_passes=False),
            mesh=mesh,
            scratch_types=[
                pltpu.VMEM((_N,), f32), pltpu.VMEM((_N,), f32),
                pltpu.VMEM((_N,), f32),
                pltpu.VMEM((_CSLICE,), f32), pltpu.VMEM((_CSLICE,), f32),
                pltpu.VMEM((_CSLICE,), f32),
                pltpu.VMEM((k,), i32), pltpu.VMEM((k,), i32),
                pltpu.VMEM((k, _CIN), f32), pltpu.VMEM((k, _CIN), f32),
                pltpu.SemaphoreType.DMA, pltpu.SemaphoreType.DMA,
            ],
        )
        outs.append(fn(px, py, pz, cx4, cy4, cz4, table_flat))
    return outs


def kernel(points_xyz, features, params):
    cx, cy, cz = _fps_pallas(points_xyz)
    cents = jnp.stack([cx, cy, cz], axis=-1)
    table = jnp.concatenate([points_xyz, features], axis=-1)
    table_flat = table.reshape(_B * _N, _CIN)
    cents_flat = cents.reshape(_B * _S, 3)

    xs = _ballquery_gather_sc(points_xyz[:, :, 0], points_xyz[:, :, 1],
                              points_xyz[:, :, 2], cx, cy, cz, table_flat)
    outs = []
    for x_raw, k, layers in zip(xs, _KS, params):
        out = _mlp_scale(x_raw, cents_flat, layers, k)
        outs.append(out.reshape(_B, _S, -1))
    return cents, jnp.concatenate(outs, axis=-1)


# speedup vs baseline: 26.2024x; 1.2104x over previous
import functools

import jax
import jax.numpy as jnp
from jax import lax
from jax.experimental import pallas as pl
from jax.experimental.pallas import tpu as pltpu
from jax.experimental.pallas import tpu_sc as plsc

_B, _N, _F = 8, 4096, 13
_S = 1024
_RADII = (0.1, 0.2, 0.4)
_KS = (16, 32, 64)
_CIN = 16
_EPS = 1e-5
_BM = 4096


def _layer1_body(x_ref, c_ref, w_ref, b_ref, y_ref, st_ref, acc_ref, *,
                 k, nblocks, total):
    i = pl.program_id(0)
    rb = _BM // k
    x = x_ref[...]
    y = jnp.dot(x, w_ref[...], preferred_element_type=jnp.float32) + b_ref[...]
    corr = jnp.dot(c_ref[...], w_ref[0:3, :], preferred_element_type=jnp.float32)
    cout = y.shape[-1]
    corr = jnp.broadcast_to(corr[:, None, :], (rb, k, cout)).reshape(_BM, cout)
    y = y - corr
    y_ref[...] = y

    @pl.when(i == 0)
    def _():
        acc_ref[...] = jnp.zeros_like(acc_ref)

    acc_ref[0:1, :] += jnp.sum(y, axis=0, keepdims=True)
    acc_ref[1:2, :] += jnp.sum(y * y, axis=0, keepdims=True)

    @pl.when(i == nblocks - 1)
    def _():
        m = acc_ref[0:1, :] / total
        st_ref[0:1, :] = m
        st_ref[1:2, :] = acc_ref[1:2, :] / total - m * m


def _layer_body(x_ref, st_in_ref, g_ref, bt_ref, w_ref, b_ref,
                y_ref, st_ref, acc_ref, *, nblocks, total):
    i = pl.program_id(0)
    mu = st_in_ref[0:1, :]
    var = st_in_ref[1:2, :]
    a = g_ref[...] * (x_ref[...] - mu) * jax.lax.rsqrt(var + _EPS) + bt_ref[...]
    a = jnp.maximum(a, 0.0)
    y = jnp.dot(a, w_ref[...], preferred_element_type=jnp.float32) + b_ref[...]
    y_ref[...] = y

    @pl.when(i == 0)
    def _():
        acc_ref[...] = jnp.zeros_like(acc_ref)

    acc_ref[0:1, :] += jnp.sum(y, axis=0, keepdims=True)
    acc_ref[1:2, :] += jnp.sum(y * y, axis=0, keepdims=True)

    @pl.when(i == nblocks - 1)
    def _():
        m = acc_ref[0:1, :] / total
        st_ref[0:1, :] = m
        st_ref[1:2, :] = acc_ref[1:2, :] / total - m * m


def _stats_body(x_ref, st_in_ref, g_ref, bt_ref, w_ref, b_ref,
                st_ref, acc_ref, *, nblocks, total):
    i = pl.program_id(0)
    mu = st_in_ref[0:1, :]
    var = st_in_ref[1:2, :]
    a = g_ref[...] * (x_ref[...] - mu) * jax.lax.rsqrt(var + _EPS) + bt_ref[...]
    a = jnp.maximum(a, 0.0)
    y = jnp.dot(a, w_ref[...], preferred_element_type=jnp.float32) + b_ref[...]

    @pl.when(i == 0)
    def _():
        acc_ref[...] = jnp.zeros_like(acc_ref)

    acc_ref[0:1, :] += jnp.sum(y, axis=0, keepdims=True)
    acc_ref[1:2, :] += jnp.sum(y * y, axis=0, keepdims=True)

    @pl.when(i == nblocks - 1)
    def _():
        m = acc_ref[0:1, :] / total
        st_ref[0:1, :] = m
        st_ref[1:2, :] = acc_ref[1:2, :] / total - m * m


def _final_body(x_ref, st2_ref, g2_ref, bt2_ref, w_ref, b_ref,
                st3_ref, g3_ref, bt3_ref, o_ref, *, k):
    mu2 = st2_ref[0:1, :]
    var2 = st2_ref[1:2, :]
    a = (g2_ref[...] * (x_ref[...] - mu2) * jax.lax.rsqrt(var2 + _EPS)
         + bt2_ref[...])
    a = jnp.maximum(a, 0.0)
    y = jnp.dot(a, w_ref[...], preferred_element_type=jnp.float32) + b_ref[...]
    mu3 = st3_ref[0:1, :]
    var3 = st3_ref[1:2, :]
    a3 = (g3_ref[...] * (y - mu3) * jax.lax.rsqrt(var3 + _EPS) + bt3_ref[...])
    a3 = jnp.maximum(a3, 0.0)
    rb = _BM // k
    c = a3.shape[-1]
    o_ref[...] = jnp.max(a3.reshape(rb, k, c), axis=1)


def _mlp_scale(x_raw, cents, layers, k):
    m = x_raw.shape[0]
    nblocks = m // _BM
    rb = _BM // k
    f32 = jnp.float32

    (w1, b1, g1, t1), (w2, b2, g2, t2), (w3, b3, g3, t3) = layers
    c1, c2, c3 = w1.shape[0], w2.shape[0], w3.shape[0]
    w1t, w2t, w3t = w1.T, w2.T, w3.T

    row_spec = lambda c: pl.BlockSpec((_BM, c), lambda i: (i, 0))
    full = lambda a: pl.BlockSpec(a.shape, lambda i: (0,) * a.ndim)

    y1, st1 = pl.pallas_call(
        functools.partial(_layer1_body, k=k, nblocks=nblocks, total=float(m)),
        grid=(nblocks,),
        in_specs=[row_spec(_CIN), pl.BlockSpec((rb, 3), lambda i: (i, 0)),
                  full(w1t), pl.BlockSpec((1, c1), lambda i: (0, 0))],
        out_specs=[row_spec(c1), pl.BlockSpec((2, c1), lambda i: (0, 0))],
        out_shape=[jax.ShapeDtypeStruct((m, c1), f32),
                   jax.ShapeDtypeStruct((2, c1), f32)],
        scratch_shapes=[pltpu.VMEM((2, c1), f32)],
    )(x_raw, cents, w1t, b1.reshape(1, c1))

    def mid(y, st, g, bt, wt, b, cin, cout):
        return pl.pallas_call(
            functools.partial(_layer_body, nblocks=nblocks, total=float(m)),
            grid=(nblocks,),
            in_specs=[row_spec(cin), pl.BlockSpec((2, cin), lambda i: (0, 0)),
                      pl.BlockSpec((1, cin), lambda i: (0, 0)),
                      pl.BlockSpec((1, cin), lambda i: (0, 0)),
                      full(wt), pl.BlockSpec((1, cout), lambda i: (0, 0))],
            out_specs=[row_spec(cout), pl.BlockSpec((2, cout), lambda i: (0, 0))],
            out_shape=[jax.ShapeDtypeStruct((m, cout), f32),
                       jax.ShapeDtypeStruct((2, cout), f32)],
            scratch_shapes=[pltpu.VMEM((2, cout), f32)],
        )(y, st, g.reshape(1, cin), bt.reshape(1, cin), wt, b.reshape(1, cout))

    y2, st2 = mid(y1, st1, g1, t1, w2t, b2, c1, c2)

    st3 = pl.pallas_call(
        functools.partial(_stats_body, nblocks=nblocks, total=float(m)),
        grid=(nblocks,),
        in_specs=[row_spec(c2), pl.BlockSpec((2, c2), lambda i: (0, 0)),
                  pl.BlockSpec((1, c2), lambda i: (0, 0)),
                  pl.BlockSpec((1, c2), lambda i: (0, 0)),
                  full(w3t), pl.BlockSpec((1, c3), lambda i: (0, 0))],
        out_specs=pl.BlockSpec((2, c3), lambda i: (0, 0)),
        out_shape=jax.ShapeDtypeStruct((2, c3), f32),
        scratch_shapes=[pltpu.VMEM((2, c3), f32)],
    )(y2, st2, g2.reshape(1, c2), t2.reshape(1, c2), w3t, b3.reshape(1, c3))

    out = pl.pallas_call(
        functools.partial(_final_body, k=k),
        grid=(nblocks,),
        in_specs=[row_spec(c2), pl.BlockSpec((2, c2), lambda i: (0, 0)),
                  pl.BlockSpec((1, c2), lambda i: (0, 0)),
                  pl.BlockSpec((1, c2), lambda i: (0, 0)),
                  full(w3t), pl.BlockSpec((1, c3), lambda i: (0, 0)),
                  pl.BlockSpec((2, c3), lambda i: (0, 0)),
                  pl.BlockSpec((1, c3), lambda i: (0, 0)),
                  pl.BlockSpec((1, c3), lambda i: (0, 0))],
        out_specs=pl.BlockSpec((rb, c3), lambda i: (i, 0)),
        out_shape=jax.ShapeDtypeStruct((m // k, c3), f32),
    )(y2, st2, g2.reshape(1, c2), t2.reshape(1, c2), w3t, b3.reshape(1, c3),
      st3, g3.reshape(1, c3), t3.reshape(1, c3))
    return out


def _fps_body(x_ref, y_ref, z_ref, cx_ref, cy_ref, cz_ref):
    x = x_ref[...]
    y = y_ref[...]
    z = z_ref[...]
    iota_n = jax.lax.broadcasted_iota(jnp.int32, (_B, _N), 1)
    iota_s = jax.lax.broadcasted_iota(jnp.int32, (_B, _S), 1)
    cx_ref[...] = jnp.zeros_like(cx_ref)
    cy_ref[...] = jnp.zeros_like(cy_ref)
    cz_ref[...] = jnp.zeros_like(cz_ref)

    def body(i, carry):
        dist, far = carry
        oh = (iota_n == far).astype(jnp.float32)
        cx = jnp.sum(x * oh, axis=1, keepdims=True)
        cy = jnp.sum(y * oh, axis=1, keepdims=True)
        cz = jnp.sum(z * oh, axis=1, keepdims=True)
        sel = (iota_s == i).astype(jnp.float32)
        cx_ref[...] += cx * sel
        cy_ref[...] += cy * sel
        cz_ref[...] += cz * sel
        dx = x - cx
        dy = y - cy
        dz = z - cz
        d = dx * dx + dy * dy + dz * dz
        dist = jnp.minimum(dist, d)
        m = jnp.max(dist, axis=1, keepdims=True)
        far2 = jnp.min(jnp.where(dist == m, iota_n, _N), axis=1, keepdims=True)
        return dist, far2.astype(jnp.int32)

    dist0 = jnp.full((_B, _N), 1e10, jnp.float32)
    far0 = jnp.zeros((_B, 1), jnp.int32)
    jax.lax.fori_loop(0, _S, body, (dist0, far0))


def _fps_pallas(points_xyz):
    f32 = jnp.float32
    x = points_xyz[:, :, 0]
    y = points_xyz[:, :, 1]
    z = points_xyz[:, :, 2]
    return pl.pallas_call(
        _fps_body,
        out_shape=[jax.ShapeDtypeStruct((_B, _S), f32)] * 3,
    )(x, y, z)


_NC, _NS = 2, 16
_CSLICE = _S // 4


def _splat(vec, lane):
    dnums = lax.GatherDimensionNumbers(
        offset_dims=(), collapsed_slice_dims=(0,), start_index_map=(0,))
    return lax.gather(vec, lane[:, None], dnums, (1,),
                      mode=lax.GatherScatterMode.PROMISE_IN_BOUNDS)


def _bq_body(px_h, py_h, pz_h, cx_h, cy_h, cz_h, table_h, x_h,
             pxv, pyv, pzv, cxv, cyv, czv,
             idx0, idx1, rows0, rows1, sem0, sem1, osem0, osem1, *, r, k):
    wid = lax.axis_index("s") * _NC + lax.axis_index("c")
    b = wid // 4
    sl = wid % 4
    pltpu.sync_copy(px_h.at[b], pxv)
    pltpu.sync_copy(py_h.at[b], pyv)
    pltpu.sync_copy(pz_h.at[b], pzv)
    pltpu.sync_copy(cx_h.at[b, sl], cxv)
    pltpu.sync_copy(cy_h.at[b, sl], cyv)
    pltpu.sync_copy(cz_h.at[b, sl], czv)
    iota = lax.iota(jnp.int32, 16)
    boff = b * _N
    ngroups = _N // 64
    r2 = r * r
    idxbs, rowss, sems = (idx0, idx1), (rows0, rows1), (sem0, sem1)
    osems = (osem0, osem1)

    def scan_pad(ci, idxb):
        lane = jnp.full((16,), ci % 16, jnp.int32)
        gbase = (ci // 16) * 16
        cxs = _splat(cxv[pl.ds(gbase, 16)], lane)
        cys = _splat(cyv[pl.ds(gbase, 16)], lane)
        czs = _splat(czv[pl.ds(gbase, 16)], lane)

        def scan_cond(c):
            return (c[0] < ngroups) & (c[1] < k)

        def scan_body(c):
            g, cnt = c
            base = g * 64
            masks = []
            for t in range(4):
                dx = pxv[pl.ds(base + t * 16, 16)] - cxs
                dy = pyv[pl.ds(base + t * 16, 16)] - cys
                dz = pzv[pl.ds(base + t * 16, 16)] - czs
                masks.append(dx * dx + dy * dy + dz * dz <= r2)
            anyv = (masks[0] | masks[1]) | (masks[2] | masks[3])

            def do_sel():
                off = jnp.full((16,), cnt, jnp.int32)
                for t in range(4):
                    cs = plsc.cumsum(masks[t].astype(jnp.int32))
                    pos = cs - 1 + off
                    wm = masks[t] & (pos < k)
                    plsc.store_scatter(idxb, [pos], iota + base + t * 16,
                                       mask=wm)
                    off = off + plsc.all_reduce_population_count(masks[t])
                return jnp.max(off)

            cnt2 = lax.cond(jnp.any(anyv), do_sel, lambda: cnt)
            return g + 1, cnt2

        _, cnt = lax.while_loop(scan_cond, scan_body, (0, 0))

        first = _splat(idxb[pl.ds(0, 16)], jnp.zeros((16,), jnp.int32))
        cntv = jnp.full((16,), cnt, jnp.int32)
        for t in range(k // 16):
            cur = idxb[pl.ds(t * 16, 16)]
            ids = iota + t * 16
            idxb[pl.ds(t * 16, 16)] = jnp.where(ids >= cntv, first, cur) + boff

    def out_base(c):
        return (b * _S + sl * _CSLICE + c) * k

    for s in (0, 1):
        scan_pad(s, idxbs[s])
        pltpu.async_copy(table_h.at[idxbs[s]], rowss[s], sems[s])

    def pair_body(p, carry):
        for s in (0, 1):
            c = 2 * p + s
            pltpu.make_async_copy(table_h.at[idxbs[s]], rowss[s],
                                  sems[s]).wait()
            pltpu.async_copy(rowss[s], x_h.at[pl.ds(out_base(c - 2), k)],
                             osems[s])
            scan_pad(c, idxbs[s])
            pltpu.make_async_copy(rowss[s],
                                  x_h.at[pl.ds(out_base(c - 2), k)],
                                  osems[s]).wait()
            pltpu.async_copy(table_h.at[idxbs[s]], rowss[s], sems[s])
        return carry

    lax.fori_loop(1, _CSLICE // 2, pair_body, 0)

    for s in (0, 1):
        pltpu.make_async_copy(table_h.at[idxbs[s]], rowss[s], sems[s]).wait()
        pltpu.sync_copy(rowss[s], x_h.at[pl.ds(out_base(_CSLICE - 2 + s), k)])


def _ballquery_gather_sc(px, py, pz, cx, cy, cz, table_flat):
    f32, i32 = jnp.float32, jnp.int32
    cx4 = cx.reshape(_B, 4, _CSLICE)
    cy4 = cy.reshape(_B, 4, _CSLICE)
    cz4 = cz.reshape(_B, 4, _CSLICE)
    outs = []
    for r, k in zip(_RADII, _KS):
        mesh = plsc.VectorSubcoreMesh(core_axis_name="c",
                                      subcore_axis_name="s")
        fn = pl.kernel(
            functools.partial(_bq_body, r=r, k=k),
            out_type=jax.ShapeDtypeStruct((_B * _S * k, _CIN), f32),
            compiler_params=pltpu.CompilerParams(
                use_tc_tiling_on_sc=False, needs_layout_passes=False),
            mesh=mesh,
            scratch_types=[
                pltpu.VMEM((_N,), f32), pltpu.VMEM((_N,), f32),
                pltpu.VMEM((_N,), f32),
                pltpu.VMEM((_CSLICE,), f32), pltpu.VMEM((_CSLICE,), f32),
                pltpu.VMEM((_CSLICE,), f32),
                pltpu.VMEM((k,), i32), pltpu.VMEM((k,), i32),
                pltpu.VMEM((k, _CIN), f32), pltpu.VMEM((k, _CIN), f32),
                pltpu.SemaphoreType.DMA, pltpu.SemaphoreType.DMA,
                pltpu.SemaphoreType.DMA, pltpu.SemaphoreType.DMA,
            ],
        )
        outs.append(fn(px, py, pz, cx4, cy4, cz4, table_flat))
    return outs


def kernel(points_xyz, features, params):
    cx, cy, cz = _fps_pallas(points_xyz)
    cents = jnp.stack([cx, cy, cz], axis=-1)
    table = jnp.concatenate([points_xyz, features], axis=-1)
    table_flat = table.reshape(_B * _N, _CIN)
    cents_flat = cents.reshape(_B * _S, 3)

    xs = _ballquery_gather_sc(points_xyz[:, :, 0], points_xyz[:, :, 1],
                              points_xyz[:, :, 2], cx, cy, cz, table_flat)
    outs = []
    for x_raw, k, layers in zip(xs, _KS, params):
        out = _mlp_scale(x_raw, cents_flat, layers, k)
        outs.append(out.reshape(_B, _S, -1))
    return cents, jnp.concatenate(outs, axis=-1)


# speedup vs baseline: 29.1867x; 1.1139x over previous
import functools

import jax
import jax.numpy as jnp
from jax import lax
from jax.experimental import pallas as pl
from jax.experimental.pallas import tpu as pltpu
from jax.experimental.pallas import tpu_sc as plsc

_B, _N, _F = 8, 4096, 13
_S = 1024
_RADII = (0.1, 0.2, 0.4)
_KS = (16, 32, 64)
_CIN = 16
_EPS = 1e-5
_BM = 8192


def _layer1_body(x_ref, c_ref, w_ref, b_ref, y_ref, st_ref, acc_ref, *,
                 k, nblocks, total):
    i = pl.program_id(0)
    rb = _BM // k
    x = x_ref[...]
    y = jnp.dot(x, w_ref[...], preferred_element_type=jnp.float32) + b_ref[...]
    corr = jnp.dot(c_ref[...], w_ref[0:3, :], preferred_element_type=jnp.float32)
    cout = y.shape[-1]
    corr = jnp.broadcast_to(corr[:, None, :], (rb, k, cout)).reshape(_BM, cout)
    y = y - corr
    y_ref[...] = y

    @pl.when(i == 0)
    def _():
        acc_ref[...] = jnp.zeros_like(acc_ref)

    acc_ref[0:1, :] += jnp.sum(y, axis=0, keepdims=True)
    acc_ref[1:2, :] += jnp.sum(y * y, axis=0, keepdims=True)

    @pl.when(i == nblocks - 1)
    def _():
        m = acc_ref[0:1, :] / total
        st_ref[0:1, :] = m
        st_ref[1:2, :] = acc_ref[1:2, :] / total - m * m


def _layer_body(x_ref, st_in_ref, g_ref, bt_ref, w_ref, b_ref,
                y_ref, st_ref, acc_ref, *, nblocks, total):
    i = pl.program_id(0)
    mu = st_in_ref[0:1, :]
    var = st_in_ref[1:2, :]
    a = g_ref[...] * (x_ref[...] - mu) * jax.lax.rsqrt(var + _EPS) + bt_ref[...]
    a = jnp.maximum(a, 0.0)
    y = jnp.dot(a, w_ref[...], preferred_element_type=jnp.float32) + b_ref[...]
    y_ref[...] = y

    @pl.when(i == 0)
    def _():
        acc_ref[...] = jnp.zeros_like(acc_ref)

    acc_ref[0:1, :] += jnp.sum(y, axis=0, keepdims=True)
    acc_ref[1:2, :] += jnp.sum(y * y, axis=0, keepdims=True)

    @pl.when(i == nblocks - 1)
    def _():
        m = acc_ref[0:1, :] / total
        st_ref[0:1, :] = m
        st_ref[1:2, :] = acc_ref[1:2, :] / total - m * m


def _stats_body(x_ref, st_in_ref, g_ref, bt_ref, w_ref, b_ref,
                st_ref, acc_ref, *, nblocks, total):
    i = pl.program_id(0)
    mu = st_in_ref[0:1, :]
    var = st_in_ref[1:2, :]
    a = g_ref[...] * (x_ref[...] - mu) * jax.lax.rsqrt(var + _EPS) + bt_ref[...]
    a = jnp.maximum(a, 0.0)
    y = jnp.dot(a, w_ref[...], preferred_element_type=jnp.float32) + b_ref[...]

    @pl.when(i == 0)
    def _():
        acc_ref[...] = jnp.zeros_like(acc_ref)

    acc_ref[0:1, :] += jnp.sum(y, axis=0, keepdims=True)
    acc_ref[1:2, :] += jnp.sum(y * y, axis=0, keepdims=True)

    @pl.when(i == nblocks - 1)
    def _():
        m = acc_ref[0:1, :] / total
        st_ref[0:1, :] = m
        st_ref[1:2, :] = acc_ref[1:2, :] / total - m * m


def _final_body(x_ref, st2_ref, g2_ref, bt2_ref, w_ref, b_ref,
                st3_ref, g3_ref, bt3_ref, o_ref, *, k):
    mu2 = st2_ref[0:1, :]
    var2 = st2_ref[1:2, :]
    a = (g2_ref[...] * (x_ref[...] - mu2) * jax.lax.rsqrt(var2 + _EPS)
         + bt2_ref[...])
    a = jnp.maximum(a, 0.0)
    y = jnp.dot(a, w_ref[...], preferred_element_type=jnp.float32) + b_ref[...]
    mu3 = st3_ref[0:1, :]
    var3 = st3_ref[1:2, :]
    a3 = (g3_ref[...] * (y - mu3) * jax.lax.rsqrt(var3 + _EPS) + bt3_ref[...])
    a3 = jnp.maximum(a3, 0.0)
    rb = _BM // k
    c = a3.shape[-1]
    o_ref[...] = jnp.max(a3.reshape(rb, k, c), axis=1)


def _mlp_scale(x_raw, cents, layers, k):
    m = x_raw.shape[0]
    nblocks = m // _BM
    rb = _BM // k
    f32 = jnp.float32

    (w1, b1, g1, t1), (w2, b2, g2, t2), (w3, b3, g3, t3) = layers
    c1, c2, c3 = w1.shape[0], w2.shape[0], w3.shape[0]
    w1t, w2t, w3t = w1.T, w2.T, w3.T

    row_spec = lambda c: pl.BlockSpec((_BM, c), lambda i: (i, 0))
    full = lambda a: pl.BlockSpec(a.shape, lambda i: (0,) * a.ndim)

    y1, st1 = pl.pallas_call(
        functools.partial(_layer1_body, k=k, nblocks=nblocks, total=float(m)),
        grid=(nblocks,),
        in_specs=[row_spec(_CIN), pl.BlockSpec((rb, 3), lambda i: (i, 0)),
                  full(w1t), pl.BlockSpec((1, c1), lambda i: (0, 0))],
        out_specs=[row_spec(c1), pl.BlockSpec((2, c1), lambda i: (0, 0))],
        out_shape=[jax.ShapeDtypeStruct((m, c1), f32),
                   jax.ShapeDtypeStruct((2, c1), f32)],
        scratch_shapes=[pltpu.VMEM((2, c1), f32)],
    )(x_raw, cents, w1t, b1.reshape(1, c1))

    def mid(y, st, g, bt, wt, b, cin, cout):
        return pl.pallas_call(
            functools.partial(_layer_body, nblocks=nblocks, total=float(m)),
            grid=(nblocks,),
            in_specs=[row_spec(cin), pl.BlockSpec((2, cin), lambda i: (0, 0)),
                      pl.BlockSpec((1, cin), lambda i: (0, 0)),
                      pl.BlockSpec((1, cin), lambda i: (0, 0)),
                      full(wt), pl.BlockSpec((1, cout), lambda i: (0, 0))],
            out_specs=[row_spec(cout), pl.BlockSpec((2, cout), lambda i: (0, 0))],
            out_shape=[jax.ShapeDtypeStruct((m, cout), f32),
                       jax.ShapeDtypeStruct((2, cout), f32)],
            scratch_shapes=[pltpu.VMEM((2, cout), f32)],
        )(y, st, g.reshape(1, cin), bt.reshape(1, cin), wt, b.reshape(1, cout))

    y2, st2 = mid(y1, st1, g1, t1, w2t, b2, c1, c2)

    st3 = pl.pallas_call(
        functools.partial(_stats_body, nblocks=nblocks, total=float(m)),
        grid=(nblocks,),
        in_specs=[row_spec(c2), pl.BlockSpec((2, c2), lambda i: (0, 0)),
                  pl.BlockSpec((1, c2), lambda i: (0, 0)),
                  pl.BlockSpec((1, c2), lambda i: (0, 0)),
                  full(w3t), pl.BlockSpec((1, c3), lambda i: (0, 0))],
        out_specs=pl.BlockSpec((2, c3), lambda i: (0, 0)),
        out_shape=jax.ShapeDtypeStruct((2, c3), f32),
        scratch_shapes=[pltpu.VMEM((2, c3), f32)],
    )(y2, st2, g2.reshape(1, c2), t2.reshape(1, c2), w3t, b3.reshape(1, c3))

    out = pl.pallas_call(
        functools.partial(_final_body, k=k),
        grid=(nblocks,),
        in_specs=[row_spec(c2), pl.BlockSpec((2, c2), lambda i: (0, 0)),
                  pl.BlockSpec((1, c2), lambda i: (0, 0)),
                  pl.BlockSpec((1, c2), lambda i: (0, 0)),
                  full(w3t), pl.BlockSpec((1, c3), lambda i: (0, 0)),
                  pl.BlockSpec((2, c3), lambda i: (0, 0)),
                  pl.BlockSpec((1, c3), lambda i: (0, 0)),
                  pl.BlockSpec((1, c3), lambda i: (0, 0))],
        out_specs=pl.BlockSpec((rb, c3), lambda i: (i, 0)),
        out_shape=jax.ShapeDtypeStruct((m // k, c3), f32),
    )(y2, st2, g2.reshape(1, c2), t2.reshape(1, c2), w3t, b3.reshape(1, c3),
      st3, g3.reshape(1, c3), t3.reshape(1, c3))
    return out


def _fps_body(x_ref, y_ref, z_ref, cx_ref, cy_ref, cz_ref):
    x = x_ref[...]
    y = y_ref[...]
    z = z_ref[...]
    iota_n = jax.lax.broadcasted_iota(jnp.int32, (_B, _N), 1)
    iota_s = jax.lax.broadcasted_iota(jnp.int32, (_B, _S), 1)
    cx_ref[...] = jnp.zeros_like(cx_ref)
    cy_ref[...] = jnp.zeros_like(cy_ref)
    cz_ref[...] = jnp.zeros_like(cz_ref)

    def body(i, carry):
        dist, far = carry
        oh = (iota_n == far).astype(jnp.float32)
        cx = jnp.sum(x * oh, axis=1, keepdims=True)
        cy = jnp.sum(y * oh, axis=1, keepdims=True)
        cz = jnp.sum(z * oh, axis=1, keepdims=True)
        sel = (iota_s == i).astype(jnp.float32)
        cx_ref[...] += cx * sel
        cy_ref[...] += cy * sel
        cz_ref[...] += cz * sel
        dx = x - cx
        dy = y - cy
        dz = z - cz
        d = dx * dx + dy * dy + dz * dz
        dist = jnp.minimum(dist, d)
        m = jnp.max(dist, axis=1, keepdims=True)
        far2 = jnp.min(jnp.where(dist == m, iota_n, _N), axis=1, keepdims=True)
        return dist, far2.astype(jnp.int32)

    dist0 = jnp.full((_B, _N), 1e10, jnp.float32)
    far0 = jnp.zeros((_B, 1), jnp.int32)
    jax.lax.fori_loop(0, _S, body, (dist0, far0))


def _fps_pallas(points_xyz):
    f32 = jnp.float32
    x = points_xyz[:, :, 0]
    y = points_xyz[:, :, 1]
    z = points_xyz[:, :, 2]
    return pl.pallas_call(
        _fps_body,
        out_shape=[jax.ShapeDtypeStruct((_B, _S), f32)] * 3,
    )(x, y, z)


_NC, _NS = 2, 16
_CSLICE = _S // 4


def _splat(vec, lane):
    dnums = lax.GatherDimensionNumbers(
        offset_dims=(), collapsed_slice_dims=(0,), start_index_map=(0,))
    return lax.gather(vec, lane[:, None], dnums, (1,),
                      mode=lax.GatherScatterMode.PROMISE_IN_BOUNDS)


def _bq_body(px_h, py_h, pz_h, cx_h, cy_h, cz_h, table_h, x_h,
             pxv, pyv, pzv, cxv, cyv, czv,
             idx0, idx1, rows0, rows1, sem0, sem1, osem0, osem1, *, r, k):
    wid = lax.axis_index("s") * _NC + lax.axis_index("c")
    b = wid // 4
    sl = wid % 4
    pltpu.sync_copy(px_h.at[b], pxv)
    pltpu.sync_copy(py_h.at[b], pyv)
    pltpu.sync_copy(pz_h.at[b], pzv)
    pltpu.sync_copy(cx_h.at[b, sl], cxv)
    pltpu.sync_copy(cy_h.at[b, sl], cyv)
    pltpu.sync_copy(cz_h.at[b, sl], czv)
    iota = lax.iota(jnp.int32, 16)
    boff = b * _N
    ngroups = _N // 64
    r2 = r * r
    idxbs, rowss, sems = (idx0, idx1), (rows0, rows1), (sem0, sem1)
    osems = (osem0, osem1)

    def scan_pad(ci, idxb):
        lane = jnp.full((16,), ci % 16, jnp.int32)
        gbase = (ci // 16) * 16
        cxs = _splat(cxv[pl.ds(gbase, 16)], lane)
        cys = _splat(cyv[pl.ds(gbase, 16)], lane)
        czs = _splat(czv[pl.ds(gbase, 16)], lane)

        def scan_cond(c):
            return (c[0] < ngroups) & (c[1] < k)

        def scan_body(c):
            g, cnt = c
            base = g * 64
            masks = []
            for t in range(4):
                dx = pxv[pl.ds(base + t * 16, 16)] - cxs
                dy = pyv[pl.ds(base + t * 16, 16)] - cys
                dz = pzv[pl.ds(base + t * 16, 16)] - czs
                masks.append(dx * dx + dy * dy + dz * dz <= r2)
            anyv = (masks[0] | masks[1]) | (masks[2] | masks[3])

            def do_sel():
                off = jnp.full((16,), cnt, jnp.int32)
                for t in range(4):
                    cs = plsc.cumsum(masks[t].astype(jnp.int32))
                    pos = cs - 1 + off
                    wm = masks[t] & (pos < k)
                    plsc.store_scatter(idxb, [pos], iota + base + t * 16,
                                       mask=wm)
                    off = off + plsc.all_reduce_population_count(masks[t])
                return jnp.max(off)

            cnt2 = lax.cond(jnp.any(anyv), do_sel, lambda: cnt)
            return g + 1, cnt2

        _, cnt = lax.while_loop(scan_cond, scan_body, (0, 0))

        first = _splat(idxb[pl.ds(0, 16)], jnp.zeros((16,), jnp.int32))
        cntv = jnp.full((16,), cnt, jnp.int32)
        for t in range(k // 16):
            cur = idxb[pl.ds(t * 16, 16)]
            ids = iota + t * 16
            idxb[pl.ds(t * 16, 16)] = jnp.where(ids >= cntv, first, cur) + boff

    def out_base(c):
        return (b * _S + sl * _CSLICE + c) * k

    for s in (0, 1):
        scan_pad(s, idxbs[s])
        pltpu.async_copy(table_h.at[idxbs[s]], rowss[s], sems[s])

    def pair_body(p, carry):
        for s in (0, 1):
            c = 2 * p + s
            pltpu.make_async_copy(table_h.at[idxbs[s]], rowss[s],
                                  sems[s]).wait()
            pltpu.async_copy(rowss[s], x_h.at[pl.ds(out_base(c - 2), k)],
                             osems[s])
            scan_pad(c, idxbs[s])
            pltpu.make_async_copy(rowss[s],
                                  x_h.at[pl.ds(out_base(c - 2), k)],
                                  osems[s]).wait()
            pltpu.async_copy(table_h.at[idxbs[s]], rowss[s], sems[s])
        return carry

    lax.fori_loop(1, _CSLICE // 2, pair_body, 0)

    for s in (0, 1):
        pltpu.make_async_copy(table_h.at[idxbs[s]], rowss[s], sems[s]).wait()
        pltpu.sync_copy(rowss[s], x_h.at[pl.ds(out_base(_CSLICE - 2 + s), k)])


def _ballquery_gather_sc(px, py, pz, cx, cy, cz, table_flat):
    f32, i32 = jnp.float32, jnp.int32
    cx4 = cx.reshape(_B, 4, _CSLICE)
    cy4 = cy.reshape(_B, 4, _CSLICE)
    cz4 = cz.reshape(_B, 4, _CSLICE)
    outs = []
    for r, k in zip(_RADII, _KS):
        mesh = plsc.VectorSubcoreMesh(core_axis_name="c",
                                      subcore_axis_name="s")
        fn = pl.kernel(
            functools.partial(_bq_body, r=r, k=k),
            out_type=jax.ShapeDtypeStruct((_B * _S * k, _CIN), f32),
            compiler_params=pltpu.CompilerParams(
                use_tc_tiling_on_sc=False, needs_layout_passes=False),
            mesh=mesh,
            scratch_types=[
                pltpu.VMEM((_N,), f32), pltpu.VMEM((_N,), f32),
                pltpu.VMEM((_N,), f32),
                pltpu.VMEM((_CSLICE,), f32), pltpu.VMEM((_CSLICE,), f32),
                pltpu.VMEM((_CSLICE,), f32),
                pltpu.VMEM((k,), i32), pltpu.VMEM((k,), i32),
                pltpu.VMEM((k, _CIN), f32), pltpu.VMEM((k, _CIN), f32),
                pltpu.SemaphoreType.DMA, pltpu.SemaphoreType.DMA,
                pltpu.SemaphoreType.DMA, pltpu.SemaphoreType.DMA,
            ],
        )
        outs.append(fn(px, py, pz, cx4, cy4, cz4, table_flat))
    return outs


def kernel(points_xyz, features, params):
    cx, cy, cz = _fps_pallas(points_xyz)
    cents = jnp.stack([cx, cy, cz], axis=-1)
    table = jnp.concatenate([points_xyz, features], axis=-1)
    table_flat = table.reshape(_B * _N, _CIN)
    cents_flat = cents.reshape(_B * _S, 3)

    xs = _ballquery_gather_sc(points_xyz[:, :, 0], points_xyz[:, :, 1],
                              points_xyz[:, :, 2], cx, cy, cz, table_flat)
    outs = []
    for x_raw, k, layers in zip(xs, _KS, params):
        out = _mlp_scale(x_raw, cents_flat, layers, k)
        outs.append(out.reshape(_B, _S, -1))
    return cents, jnp.concatenate(outs, axis=-1)
